# baseline (speedup 1.0000x reference)
import functools

import jax
import jax.numpy as jnp
from jax import lax
from jax.experimental import pallas as pl
from jax.experimental.pallas import tpu as pltpu
from jax.experimental.pallas import tpu_sc as plsc

N = 10000
E = 320000

NC = 2
NS = 16
NW = NC * NS
WLEN = 128
RW = 80
E_PAD = NW * RW * WLEN
N_A = 10240
RPT = N_A // NS
NCH = RPT // WLEN

_mesh = functools.partial(plsc.VectorSubcoreMesh,
                          core_axis_name="c", subcore_axis_name="s")
_SC_PARAMS = pltpu.CompilerParams(use_tc_tiling_on_sc=False)


def _worker(c, s):
    return c * NS + s


def _sc_degrees(src2d, dst2d):
    @functools.partial(
        pl.kernel, mesh=_mesh(), compiler_params=_SC_PARAMS,
        out_type=[jax.ShapeDtypeStruct((N_A,), jnp.float32),
                  jax.ShapeDtypeStruct((N_A,), jnp.float32),
                  jax.ShapeDtypeStruct((N_A,), jnp.float32),
                  jax.ShapeDtypeStruct((N_A,), jnp.float32)],
        scratch_types=[
            pltpu.VMEM((RW, 1, WLEN), jnp.int32),
            pltpu.VMEM((RW, 1, WLEN), jnp.int32),
            pltpu.VMEM((WLEN,), jnp.float32),
            pltpu.VMEM((WLEN,), jnp.float32),
            pltpu.VMEM_SHARED((N_A,), jnp.float32),
            pltpu.VMEM_SHARED((N_A,), jnp.float32),
        ])
    def k(s_hbm, d_hbm, do0_hbm, do1_hbm, di0_hbm, di1_hbm,
          sidx, didx, ones_v, stg_v, acc_o, acc_i):
        c = lax.axis_index("c")
        s = lax.axis_index("s")

        @pl.loop(0, WLEN // 16)
        def _(i):
            ones_v[pl.ds(i * 16, 16)] = jnp.full((16,), 1.0, jnp.float32)
            stg_v[pl.ds(i * 16, 16)] = jnp.zeros((16,), jnp.float32)

        @pl.loop(0, NCH)
        def _(kk):
            sl = pl.ds(s * RPT + kk * WLEN, WLEN)
            pltpu.sync_copy(stg_v, acc_o.at[sl])
            pltpu.sync_copy(stg_v, acc_i.at[sl])

        w = _worker(c, s)
        pltpu.sync_copy(s_hbm.at[pl.ds(w * RW, RW)], sidx)
        pltpu.sync_copy(d_hbm.at[pl.ds(w * RW, RW)], didx)
        plsc.subcore_barrier()

        @pl.loop(0, RW)
        def _(r):
            pltpu.sync_copy(ones_v, acc_o.at[sidx.at[r, 0]], add=True)
            pltpu.sync_copy(ones_v, acc_i.at[didx.at[r, 0]], add=True)

        plsc.subcore_barrier()

        @pl.loop(0, NCH)
        def _(kk):
            sl = pl.ds(s * RPT + kk * WLEN, WLEN)

            @pl.when(c == 0)
            def _():
                pltpu.sync_copy(acc_o.at[sl], stg_v)
                pltpu.sync_copy(stg_v, do0_hbm.at[sl])
                pltpu.sync_copy(acc_i.at[sl], stg_v)
                pltpu.sync_copy(stg_v, di0_hbm.at[sl])

            @pl.when(c == 1)
            def _():
                pltpu.sync_copy(acc_o.at[sl], stg_v)
                pltpu.sync_copy(stg_v, do1_hbm.at[sl])
                pltpu.sync_copy(acc_i.at[sl], stg_v)
                pltpu.sync_copy(stg_v, di1_hbm.at[sl])

    return k(src2d, dst2d)


def _sc_gcn_agg(h48, src2d, dst2d):
    @functools.partial(
        pl.kernel, mesh=_mesh(), compiler_params=_SC_PARAMS,
        out_type=jax.ShapeDtypeStruct((NC, N_A, 48), jnp.float32),
        scratch_types=[
            pltpu.VMEM((RW, 1, WLEN), jnp.int32),
            pltpu.VMEM((RW, 1, WLEN), jnp.int32),
            pltpu.VMEM((WLEN, 48), jnp.float32),
            pltpu.VMEM_SHARED((N_A, 48), jnp.float32),
        ])
    def k(h_hbm, s_hbm, d_hbm, o_hbm, sidx, didx, g_v, acc):
        c = lax.axis_index("c")
        s = lax.axis_index("s")

        @pl.loop(0, WLEN)
        def _(r):
            for j in range(3):
                g_v[r, pl.ds(j * 16, 16)] = jnp.zeros((16,), jnp.float32)

        @pl.loop(0, NCH)
        def _(kk):
            pltpu.sync_copy(g_v, acc.at[pl.ds(s * RPT + kk * WLEN, WLEN)])

        w = _worker(c, s)
        pltpu.sync_copy(s_hbm.at[pl.ds(w * RW, RW)], sidx)
        pltpu.sync_copy(d_hbm.at[pl.ds(w * RW, RW)], didx)
        plsc.subcore_barrier()

        @pl.loop(0, RW)
        def _(r):
            pltpu.sync_copy(h_hbm.at[sidx.at[r, 0]], g_v)
            pltpu.sync_copy(g_v, acc.at[didx.at[r, 0]], add=True)

        plsc.subcore_barrier()

        @pl.loop(0, NCH)
        def _(kk):
            sl = pl.ds(s * RPT + kk * WLEN, WLEN)
            pltpu.sync_copy(acc.at[sl], g_v)
            pltpu.sync_copy(g_v, o_hbm.at[c, sl])

    return k(h48, src2d, dst2d)


def _graph_conv(x, W, b, src2d, dst2d, norm_src, norm_dst):
    h = (x * norm_src) @ W
    h48 = jnp.pad(h, ((0, 0), (0, 12)))
    aggp = _sc_gcn_agg(h48, src2d, dst2d)
    agg = (aggp[0] + aggp[1])[:, :36]
    return agg * norm_dst + b


def _gat_conv(x, W, A_el, A_er, b, src, dst, heads, out_dim):
    feat = x @ W
    el = feat @ A_el
    er = feat @ A_er
    e = jax.nn.leaky_relu(el[src] + er[dst], negative_slope=0.2)
    emax = jax.ops.segment_max(e, dst, num_segments=N_A)
    emax = jnp.where(jnp.isfinite(emax), emax, 0.0)
    ee = jnp.exp(e - emax[dst])
    denom = jax.ops.segment_sum(ee, dst, num_segments=N_A)
    alpha = ee / (denom[dst] + 1e-9)
    feat3 = feat.reshape(N_A, heads, out_dim)
    msg = feat3[src] * alpha[:, :, None]
    rst = jax.ops.segment_sum(msg, dst, num_segments=N_A)
    return (rst + b.reshape(1, heads, out_dim)).reshape(N_A, heads * out_dim)


def _head_matrix(a):
    heads, od = a.shape
    idx = jnp.arange(heads * od)
    return jnp.zeros((heads * od, heads), jnp.float32).at[idx, idx // od].set(
        a.reshape(-1))


def kernel(features, gca1_gcn_W, gca1_gcn_b, gca1_gat_W, gca1_gat_al,
           gca1_gat_ar, gca1_gat_b, gca_gcn_W, gca_gcn_b, gca_gat_W,
           gca_gat_al, gca_gat_ar, gca_gat_b, ma_W, ma_al, ma_ar, ma_b,
           dense_W, dense_b, edge_index, num_blocks_Q, num_blocks_L):
    src = edge_index[0].astype(jnp.int32)
    dst = edge_index[1].astype(jnp.int32)
    npad = E_PAD - E
    pad_ids = (N + (jnp.arange(npad, dtype=jnp.int32) % 32)).astype(jnp.int32)
    src2d = jnp.concatenate([src, pad_ids]).reshape(NW * RW, 1, WLEN)
    dst2d = jnp.concatenate([dst, pad_ids]).reshape(NW * RW, 1, WLEN)

    do0, do1, di0, di1 = _sc_degrees(src2d, dst2d)
    deg_out = do0 + do1
    deg_in = di0 + di1
    norm_src = (jnp.where(deg_out > 0, deg_out, 1.0) ** -0.5)[:, None]
    norm_dst = (jnp.where(deg_in > 0, deg_in, 1.0) ** -0.5)[:, None]

    x0 = jnp.pad(features, ((0, N_A - N), (0, 0)))
    srcp = jnp.concatenate([src, pad_ids])
    dstp = jnp.concatenate([dst, pad_ids])

    A1_el = _head_matrix(gca1_gat_al)
    A1_er = _head_matrix(gca1_gat_ar)
    A_el = _head_matrix(gca_gat_al)
    A_er = _head_matrix(gca_gat_ar)
    Am_el = _head_matrix(ma_al)
    Am_er = _head_matrix(ma_ar)

    def gca1(x):
        h = _graph_conv(x, gca1_gcn_W, gca1_gcn_b, src2d, dst2d,
                        norm_src, norm_dst)
        return _gat_conv(h, gca1_gat_W, A1_el, A1_er, gca1_gat_b,
                         srcp, dstp, 6, 6)

    def gca(x):
        h = _graph_conv(x, gca_gcn_W, gca_gcn_b, src2d, dst2d,
                        norm_src, norm_dst)
        return _gat_conv(h, gca_gat_W, A_el, A_er, gca_gat_b, srcp, dstp, 6, 6)

    def _residual_block(i, x):
        return x + gca(gca(x))

    x = gca1(x0)
    x = jax.lax.fori_loop(0, num_blocks_Q // 2, _residual_block, x)
    x = _gat_conv(x, ma_W, Am_el, Am_er, ma_b, srcp, dstp, 6, 32)
    x = gca1(x)
    x = jax.lax.fori_loop(0, num_blocks_L // 2, _residual_block, x)
    return x[:N] @ dense_W + dense_b


# speedup vs baseline: 53.6334x; 53.6334x over previous
import functools

import jax
import jax.numpy as jnp
from jax import lax
from jax.experimental import pallas as pl
from jax.experimental.pallas import tpu as pltpu
from jax.experimental.pallas import tpu_sc as plsc

N = 10000
E = 320000

NC = 2
NS = 16
NW = NC * NS
WLEN = 128
RW = 80
E_PAD = NW * RW * WLEN
N_A = 10240
RPT = N_A // NS
NCH = RPT // WLEN

_mesh = functools.partial(plsc.VectorSubcoreMesh,
                          core_axis_name="c", subcore_axis_name="s")
_SC_PARAMS = pltpu.CompilerParams(use_tc_tiling_on_sc=False,
                                  needs_layout_passes=False)


def _worker(c, s):
    return c * NS + s


def _sc_degrees(src2d, dst2d):
    @functools.partial(
        pl.kernel, mesh=_mesh(), compiler_params=_SC_PARAMS,
        out_type=[jax.ShapeDtypeStruct((N_A,), jnp.float32),
                  jax.ShapeDtypeStruct((N_A,), jnp.float32),
                  jax.ShapeDtypeStruct((N_A,), jnp.float32),
                  jax.ShapeDtypeStruct((N_A,), jnp.float32)],
        scratch_types=[
            pltpu.VMEM((RW, 1, WLEN), jnp.int32),
            pltpu.VMEM((RW, 1, WLEN), jnp.int32),
            pltpu.VMEM((WLEN,), jnp.float32),
            pltpu.VMEM((WLEN,), jnp.float32),
            pltpu.VMEM_SHARED((N_A,), jnp.float32),
            pltpu.VMEM_SHARED((N_A,), jnp.float32),
        ])
    def k(s_hbm, d_hbm, do0_hbm, do1_hbm, di0_hbm, di1_hbm,
          sidx, didx, ones_v, stg_v, acc_o, acc_i):
        c = lax.axis_index("c")
        s = lax.axis_index("s")

        @pl.loop(0, WLEN // 16)
        def _(i):
            ones_v[pl.ds(i * 16, 16)] = jnp.full((16,), 1.0, jnp.float32)
            stg_v[pl.ds(i * 16, 16)] = jnp.zeros((16,), jnp.float32)

        @pl.loop(0, NCH)
        def _(kk):
            sl = pl.ds(s * RPT + kk * WLEN, WLEN)
            pltpu.sync_copy(stg_v, acc_o.at[sl])
            pltpu.sync_copy(stg_v, acc_i.at[sl])

        w = _worker(c, s)
        pltpu.sync_copy(s_hbm.at[pl.ds(w * RW, RW)], sidx)
        pltpu.sync_copy(d_hbm.at[pl.ds(w * RW, RW)], didx)
        plsc.subcore_barrier()

        @pl.loop(0, RW)
        def _(r):
            pltpu.sync_copy(ones_v, acc_o.at[sidx.at[r, 0]], add=True)
            pltpu.sync_copy(ones_v, acc_i.at[didx.at[r, 0]], add=True)

        plsc.subcore_barrier()

        @pl.loop(0, NCH)
        def _(kk):
            sl = pl.ds(s * RPT + kk * WLEN, WLEN)

            @pl.when(c == 0)
            def _():
                pltpu.sync_copy(acc_o.at[sl], stg_v)
                pltpu.sync_copy(stg_v, do0_hbm.at[sl])
                pltpu.sync_copy(acc_i.at[sl], stg_v)
                pltpu.sync_copy(stg_v, di0_hbm.at[sl])

            @pl.when(c == 1)
            def _():
                pltpu.sync_copy(acc_o.at[sl], stg_v)
                pltpu.sync_copy(stg_v, do1_hbm.at[sl])
                pltpu.sync_copy(acc_i.at[sl], stg_v)
                pltpu.sync_copy(stg_v, di1_hbm.at[sl])

    return k(src2d, dst2d)


def _sc_gcn_agg(h48, src2d, dst2d):
    @functools.partial(
        pl.kernel, mesh=_mesh(), compiler_params=_SC_PARAMS,
        out_type=jax.ShapeDtypeStruct((NC, N_A, 48), jnp.float32),
        scratch_types=[
            pltpu.VMEM((RW, 1, WLEN), jnp.int32),
            pltpu.VMEM((RW, 1, WLEN), jnp.int32),
            pltpu.VMEM((WLEN, 48), jnp.float32),
            pltpu.VMEM_SHARED((N_A, 48), jnp.float32),
        ])
    def k(h_hbm, s_hbm, d_hbm, o_hbm, sidx, didx, g_v, acc):
        c = lax.axis_index("c")
        s = lax.axis_index("s")

        @pl.loop(0, WLEN)
        def _(r):
            for j in range(3):
                g_v[r, pl.ds(j * 16, 16)] = jnp.zeros((16,), jnp.float32)

        @pl.loop(0, NCH)
        def _(kk):
            pltpu.sync_copy(g_v, acc.at[pl.ds(s * RPT + kk * WLEN, WLEN)])

        w = _worker(c, s)
        pltpu.sync_copy(s_hbm.at[pl.ds(w * RW, RW)], sidx)
        pltpu.sync_copy(d_hbm.at[pl.ds(w * RW, RW)], didx)
        plsc.subcore_barrier()

        @pl.loop(0, RW)
        def _(r):
            pltpu.sync_copy(h_hbm.at[sidx.at[r, 0]], g_v)
            pltpu.sync_copy(g_v, acc.at[didx.at[r, 0]], add=True)

        plsc.subcore_barrier()

        @pl.loop(0, NCH)
        def _(kk):
            sl = pl.ds(s * RPT + kk * WLEN, WLEN)
            pltpu.sync_copy(acc.at[sl], g_v)
            pltpu.sync_copy(g_v, o_hbm.at[c, sl])

    return k(h48, src2d, dst2d)


def _sc_gat_denom(el16, er16, src2d, dst2d):
    @functools.partial(
        pl.kernel, mesh=_mesh(), compiler_params=_SC_PARAMS,
        out_type=jax.ShapeDtypeStruct((NC, N_A, 16), jnp.float32),
        scratch_types=[
            pltpu.VMEM((RW, 1, WLEN), jnp.int32),
            pltpu.VMEM((RW, 1, WLEN), jnp.int32),
            pltpu.VMEM((WLEN, 16), jnp.float32),
            pltpu.VMEM((WLEN, 16), jnp.float32),
            pltpu.VMEM((WLEN, 16), jnp.float32),
            pltpu.VMEM_SHARED((N_A, 16), jnp.float32),
        ])
    def k(el_hbm, er_hbm, s_hbm, d_hbm, o_hbm, sidx, didx, elg, erg, eev, acc):
        c = lax.axis_index("c")
        s = lax.axis_index("s")

        @pl.loop(0, WLEN)
        def _(r):
            eev[r, :] = jnp.zeros((16,), jnp.float32)

        @pl.loop(0, NCH)
        def _(kk):
            pltpu.sync_copy(eev, acc.at[pl.ds(s * RPT + kk * WLEN, WLEN)])

        w = _worker(c, s)
        pltpu.sync_copy(s_hbm.at[pl.ds(w * RW, RW)], sidx)
        pltpu.sync_copy(d_hbm.at[pl.ds(w * RW, RW)], didx)
        plsc.subcore_barrier()

        @pl.loop(0, RW)
        def _(r):
            pltpu.sync_copy(el_hbm.at[sidx.at[r, 0]], elg)
            pltpu.sync_copy(er_hbm.at[didx.at[r, 0]], erg)

            @pl.loop(0, WLEN // 16)
            def _(ch):
                rows = ch * 16 + lax.iota(jnp.int32, 16)
                for h in range(6):
                    col = jnp.full((16,), h, jnp.int32)
                    z = (plsc.load_gather(elg, [rows, col])
                         + plsc.load_gather(erg, [rows, col]))
                    z = jnp.maximum(z, 0.2 * z)
                    plsc.store_scatter(eev, [rows, col], jnp.exp(z))

            pltpu.sync_copy(eev, acc.at[didx.at[r, 0]], add=True)

        plsc.subcore_barrier()

        @pl.loop(0, NCH)
        def _(kk):
            sl = pl.ds(s * RPT + kk * WLEN, WLEN)
            pltpu.sync_copy(acc.at[sl], eev)
            pltpu.sync_copy(eev, o_hbm.at[c, sl])

    return k(el16, er16, src2d, dst2d)


def _sc_gat_msg(G, erd16, src2d, dst2d, el16=None, elcol=36, od=6, col0=0):
    GW = G.shape[1]
    nk = GW // 16
    sep_el = el16 is not None
    scratch = [
        pltpu.VMEM((RW, 1, WLEN), jnp.int32),
        pltpu.VMEM((RW, 1, WLEN), jnp.int32),
        pltpu.VMEM((WLEN, GW), jnp.float32),
        pltpu.VMEM((WLEN, 16), jnp.float32),
        pltpu.VMEM((WLEN, 16), jnp.float32),
        pltpu.VMEM_SHARED((N_A, GW), jnp.float32),
    ]
    if sep_el:
        scratch.insert(4, pltpu.VMEM((WLEN, 16), jnp.float32))

    def body(G_hbm, erd_hbm, s_hbm, d_hbm, el_hbm, o_hbm,
             sidx, didx, g, erd, elg, al, acc):
        c = lax.axis_index("c")
        s = lax.axis_index("s")

        @pl.loop(0, WLEN)
        def _(r):
            for j in range(nk):
                g[r, pl.ds(j * 16, 16)] = jnp.zeros((16,), jnp.float32)
            al[r, :] = jnp.zeros((16,), jnp.float32)

        @pl.loop(0, NCH)
        def _(kk):
            pltpu.sync_copy(g, acc.at[pl.ds(s * RPT + kk * WLEN, WLEN)])

        w = _worker(c, s)
        pltpu.sync_copy(s_hbm.at[pl.ds(w * RW, RW)], sidx)
        pltpu.sync_copy(d_hbm.at[pl.ds(w * RW, RW)], didx)
        plsc.subcore_barrier()

        headmaps = [(lax.iota(jnp.int32, 16) + (col0 + 16 * k)) // od
                    for k in range(nk)]

        @pl.loop(0, RW)
        def _(r):
            pltpu.sync_copy(G_hbm.at[sidx.at[r, 0]], g)
            pltpu.sync_copy(erd_hbm.at[didx.at[r, 0]], erd)
            if sep_el:
                pltpu.sync_copy(el_hbm.at[sidx.at[r, 0]], elg)

            @pl.loop(0, WLEN // 16)
            def _(ch):
                rows = ch * 16 + lax.iota(jnp.int32, 16)
                for h in range(6):
                    col = jnp.full((16,), h, jnp.int32)
                    if sep_el:
                        elv = plsc.load_gather(elg, [rows, col])
                    else:
                        elv = plsc.load_gather(
                            g, [rows, jnp.full((16,), elcol + h, jnp.int32)])
                    erv = plsc.load_gather(erd, [rows, col])
                    dnv = plsc.load_gather(
                        erd, [rows, jnp.full((16,), 6 + h, jnp.int32)])
                    z = elv + erv
                    z = jnp.maximum(z, 0.2 * z)
                    alpha = jnp.exp(z) / (dnv + 1e-9)
                    plsc.store_scatter(al, [rows, col], alpha)

            @pl.loop(0, WLEN)
            def _(e):
                erow = jnp.full((16,), e, jnp.int32)
                for k in range(nk):
                    av = plsc.load_gather(al, [erow, headmaps[k]])
                    g[e, pl.ds(16 * k, 16)] = g[e, pl.ds(16 * k, 16)] * av

            pltpu.sync_copy(g, acc.at[didx.at[r, 0]], add=True)

        plsc.subcore_barrier()

        @pl.loop(0, NCH)
        def _(kk):
            sl = pl.ds(s * RPT + kk * WLEN, WLEN)
            pltpu.sync_copy(acc.at[sl], g)
            pltpu.sync_copy(g, o_hbm.at[c, sl])

    if sep_el:
        @functools.partial(
            pl.kernel, mesh=_mesh(), compiler_params=_SC_PARAMS,
            out_type=jax.ShapeDtypeStruct((NC, N_A, GW), jnp.float32),
            scratch_types=scratch)
        def k(G_hbm, erd_hbm, s_hbm, d_hbm, el_hbm, o_hbm,
              sidx, didx, g, erd, elg, al, acc):
            body(G_hbm, erd_hbm, s_hbm, d_hbm, el_hbm, o_hbm,
                 sidx, didx, g, erd, elg, al, acc)
        return k(G, erd16, src2d, dst2d, el16)
    else:
        @functools.partial(
            pl.kernel, mesh=_mesh(), compiler_params=_SC_PARAMS,
            out_type=jax.ShapeDtypeStruct((NC, N_A, GW), jnp.float32),
            scratch_types=scratch)
        def k(G_hbm, erd_hbm, s_hbm, d_hbm, o_hbm,
              sidx, didx, g, erd, al, acc):
            body(G_hbm, erd_hbm, s_hbm, d_hbm, None, o_hbm,
                 sidx, didx, g, erd, None, al, acc)
        return k(G, erd16, src2d, dst2d)


def _graph_conv(x, W, b, src2d, dst2d, norm_src, norm_dst):
    h = (x * norm_src) @ W
    h48 = jnp.pad(h, ((0, 0), (0, 12)))
    aggp = _sc_gcn_agg(h48, src2d, dst2d)
    agg = (aggp[0] + aggp[1])[:, :36]
    return agg * norm_dst + b


def _gat_conv(x, W, A_el, A_er, b, src2d, dst2d, out_dim):
    feat = x @ W
    el = feat @ A_el
    er = feat @ A_er
    el16 = jnp.pad(el, ((0, 0), (0, 10)))
    er16 = jnp.pad(er, ((0, 0), (0, 10)))
    denp = _sc_gat_denom(el16, er16, src2d, dst2d)
    denom = (denp[0] + denp[1])[:, :6]
    erd16 = jnp.concatenate([er, denom, jnp.zeros((N_A, 4), jnp.float32)], 1)
    if out_dim == 6:
        G = jnp.concatenate([feat, el, jnp.zeros((N_A, 6), jnp.float32)], 1)
        rstp = _sc_gat_msg(G, erd16, src2d, dst2d, od=6)
    else:
        rstpA = _sc_gat_msg(feat[:, :96], erd16, src2d, dst2d,
                            el16=el16, od=out_dim, col0=0)
        rstpB = _sc_gat_msg(feat[:, 96:], erd16, src2d, dst2d,
                            el16=el16, od=out_dim, col0=96)
        rstp = jnp.concatenate([rstpA, rstpB], axis=2)
    rst = (rstp[0] + rstp[1])[:, :6 * out_dim]
    return rst + b.reshape(1, 6 * out_dim)


def _head_matrix(a):
    heads, od = a.shape
    idx = jnp.arange(heads * od)
    return jnp.zeros((heads * od, heads), jnp.float32).at[idx, idx // od].set(
        a.reshape(-1))


def kernel(features, gca1_gcn_W, gca1_gcn_b, gca1_gat_W, gca1_gat_al,
           gca1_gat_ar, gca1_gat_b, gca_gcn_W, gca_gcn_b, gca_gat_W,
           gca_gat_al, gca_gat_ar, gca_gat_b, ma_W, ma_al, ma_ar, ma_b,
           dense_W, dense_b, edge_index, num_blocks_Q, num_blocks_L):
    src = edge_index[0].astype(jnp.int32)
    dst = edge_index[1].astype(jnp.int32)
    npad = E_PAD - E
    pad_ids = (N + (jnp.arange(npad, dtype=jnp.int32) % 32)).astype(jnp.int32)
    src2d = jnp.concatenate([src, pad_ids]).reshape(NW * RW, 1, WLEN)
    dst2d = jnp.concatenate([dst, pad_ids]).reshape(NW * RW, 1, WLEN)

    do0, do1, di0, di1 = _sc_degrees(src2d, dst2d)
    deg_out = do0 + do1
    deg_in = di0 + di1
    norm_src = (jnp.where(deg_out > 0, deg_out, 1.0) ** -0.5)[:, None]
    norm_dst = (jnp.where(deg_in > 0, deg_in, 1.0) ** -0.5)[:, None]

    x0 = jnp.pad(features, ((0, N_A - N), (0, 0)))

    A1_el = _head_matrix(gca1_gat_al)
    A1_er = _head_matrix(gca1_gat_ar)
    A_el = _head_matrix(gca_gat_al)
    A_er = _head_matrix(gca_gat_ar)
    Am_el = _head_matrix(ma_al)
    Am_er = _head_matrix(ma_ar)

    def gca1(x):
        h = _graph_conv(x, gca1_gcn_W, gca1_gcn_b, src2d, dst2d,
                        norm_src, norm_dst)
        return _gat_conv(h, gca1_gat_W, A1_el, A1_er, gca1_gat_b,
                         src2d, dst2d, 6)

    def gca(x):
        h = _graph_conv(x, gca_gcn_W, gca_gcn_b, src2d, dst2d,
                        norm_src, norm_dst)
        return _gat_conv(h, gca_gat_W, A_el, A_er, gca_gat_b, src2d, dst2d, 6)

    def _residual_block(i, x):
        return x + gca(gca(x))

    x = gca1(x0)
    x = jax.lax.fori_loop(0, num_blocks_Q // 2, _residual_block, x)
    x = _gat_conv(x, ma_W, Am_el, Am_er, ma_b, src2d, dst2d, 32)
    x = gca1(x)
    x = jax.lax.fori_loop(0, num_blocks_L // 2, _residual_block, x)
    return x[:N] @ dense_W + dense_b


# speedup vs baseline: 69.1493x; 1.2893x over previous
import functools

import jax
import jax.numpy as jnp
from jax import lax
from jax.experimental import pallas as pl
from jax.experimental.pallas import tpu as pltpu
from jax.experimental.pallas import tpu_sc as plsc

N = 10000
E = 320000

NC = 2
NS = 16
NW = NC * NS
WLEN = 128
RW = 80
E_PAD = NW * RW * WLEN
N_A = 10240
RPT = N_A // NS
NCH = RPT // WLEN

_mesh = functools.partial(plsc.VectorSubcoreMesh,
                          core_axis_name="c", subcore_axis_name="s")
_SC_PARAMS = pltpu.CompilerParams(use_tc_tiling_on_sc=False,
                                  needs_layout_passes=False)


def _worker(c, s):
    return c * NS + s


def _sc_degrees(src2d, dst2d):
    @functools.partial(
        pl.kernel, mesh=_mesh(), compiler_params=_SC_PARAMS,
        out_type=[jax.ShapeDtypeStruct((N_A,), jnp.float32),
                  jax.ShapeDtypeStruct((N_A,), jnp.float32),
                  jax.ShapeDtypeStruct((N_A,), jnp.float32),
                  jax.ShapeDtypeStruct((N_A,), jnp.float32)],
        scratch_types=[
            pltpu.VMEM((RW, 1, WLEN), jnp.int32),
            pltpu.VMEM((RW, 1, WLEN), jnp.int32),
            pltpu.VMEM((WLEN,), jnp.float32),
            pltpu.VMEM((WLEN,), jnp.float32),
            pltpu.VMEM_SHARED((N_A,), jnp.float32),
            pltpu.VMEM_SHARED((N_A,), jnp.float32),
        ])
    def k(s_hbm, d_hbm, do0_hbm, do1_hbm, di0_hbm, di1_hbm,
          sidx, didx, ones_v, stg_v, acc_o, acc_i):
        c = lax.axis_index("c")
        s = lax.axis_index("s")

        @pl.loop(0, WLEN // 16)
        def _(i):
            ones_v[pl.ds(i * 16, 16)] = jnp.full((16,), 1.0, jnp.float32)
            stg_v[pl.ds(i * 16, 16)] = jnp.zeros((16,), jnp.float32)

        @pl.loop(0, NCH)
        def _(kk):
            sl = pl.ds(s * RPT + kk * WLEN, WLEN)
            pltpu.sync_copy(stg_v, acc_o.at[sl])
            pltpu.sync_copy(stg_v, acc_i.at[sl])

        w = _worker(c, s)
        pltpu.sync_copy(s_hbm.at[pl.ds(w * RW, RW)], sidx)
        pltpu.sync_copy(d_hbm.at[pl.ds(w * RW, RW)], didx)
        plsc.subcore_barrier()

        @pl.loop(0, RW)
        def _(r):
            pltpu.sync_copy(ones_v, acc_o.at[sidx.at[r, 0]], add=True)
            pltpu.sync_copy(ones_v, acc_i.at[didx.at[r, 0]], add=True)

        plsc.subcore_barrier()

        @pl.loop(0, NCH)
        def _(kk):
            sl = pl.ds(s * RPT + kk * WLEN, WLEN)

            @pl.when(c == 0)
            def _():
                pltpu.sync_copy(acc_o.at[sl], stg_v)
                pltpu.sync_copy(stg_v, do0_hbm.at[sl])
                pltpu.sync_copy(acc_i.at[sl], stg_v)
                pltpu.sync_copy(stg_v, di0_hbm.at[sl])

            @pl.when(c == 1)
            def _():
                pltpu.sync_copy(acc_o.at[sl], stg_v)
                pltpu.sync_copy(stg_v, do1_hbm.at[sl])
                pltpu.sync_copy(acc_i.at[sl], stg_v)
                pltpu.sync_copy(stg_v, di1_hbm.at[sl])

    return k(src2d, dst2d)


def _sc_gcn_agg(h48, src2d, dst2d):
    @functools.partial(
        pl.kernel, mesh=_mesh(), compiler_params=_SC_PARAMS,
        out_type=jax.ShapeDtypeStruct((NC, N_A, 48), jnp.float32),
        scratch_types=[
            pltpu.VMEM((RW, 1, WLEN), jnp.int32),
            pltpu.VMEM((RW, 1, WLEN), jnp.int32),
            pltpu.VMEM((WLEN, 48), jnp.float32),
            pltpu.VMEM_SHARED((N_A, 48), jnp.float32),
        ])
    def k(h_hbm, s_hbm, d_hbm, o_hbm, sidx, didx, g_v, acc):
        c = lax.axis_index("c")
        s = lax.axis_index("s")

        @pl.loop(0, WLEN)
        def _(r):
            for j in range(3):
                g_v[r, pl.ds(j * 16, 16)] = jnp.zeros((16,), jnp.float32)

        @pl.loop(0, NCH)
        def _(kk):
            pltpu.sync_copy(g_v, acc.at[pl.ds(s * RPT + kk * WLEN, WLEN)])

        w = _worker(c, s)
        pltpu.sync_copy(s_hbm.at[pl.ds(w * RW, RW)], sidx)
        pltpu.sync_copy(d_hbm.at[pl.ds(w * RW, RW)], didx)
        plsc.subcore_barrier()

        @pl.loop(0, RW)
        def _(r):
            pltpu.sync_copy(h_hbm.at[sidx.at[r, 0]], g_v)
            pltpu.sync_copy(g_v, acc.at[didx.at[r, 0]], add=True)

        plsc.subcore_barrier()

        @pl.loop(0, NCH)
        def _(kk):
            sl = pl.ds(s * RPT + kk * WLEN, WLEN)
            pltpu.sync_copy(acc.at[sl], g_v)
            pltpu.sync_copy(g_v, o_hbm.at[c, sl])

    return k(h48, src2d, dst2d)


def _sc_gat_fused(G, er16, src2d, dst2d, el16=None, elcol=36, od=6, col0=0,
                  with_denom=True):
    GW = G.shape[1]
    nk = GW // 16
    sep_el = el16 is not None
    scratch = [
        pltpu.VMEM((RW, 1, WLEN), jnp.int32),
        pltpu.VMEM((RW, 1, WLEN), jnp.int32),
        pltpu.VMEM((WLEN, GW), jnp.float32),
        pltpu.VMEM((WLEN, 16), jnp.float32),
        pltpu.VMEM((WLEN, 16), jnp.float32),
        pltpu.VMEM_SHARED((N_A, GW), jnp.float32),
    ]
    if sep_el:
        scratch.insert(4, pltpu.VMEM((WLEN, 16), jnp.float32))
    out_type = [jax.ShapeDtypeStruct((NC, N_A, GW), jnp.float32)]
    if with_denom:
        scratch.append(pltpu.VMEM_SHARED((N_A, 16), jnp.float32))
        out_type.append(jax.ShapeDtypeStruct((NC, N_A, 16), jnp.float32))

    def body(G_hbm, er_hbm, s_hbm, d_hbm, el_hbm, o_hbm, od_hbm,
             sidx, didx, g, erg, elg, al, acc, dacc):
        c = lax.axis_index("c")
        s = lax.axis_index("s")

        @pl.loop(0, WLEN)
        def _(r):
            for j in range(nk):
                g[r, pl.ds(j * 16, 16)] = jnp.zeros((16,), jnp.float32)
            al[r, :] = jnp.zeros((16,), jnp.float32)

        @pl.loop(0, NCH)
        def _(kk):
            pltpu.sync_copy(g, acc.at[pl.ds(s * RPT + kk * WLEN, WLEN)])
            if dacc is not None:
                pltpu.sync_copy(al, dacc.at[pl.ds(s * RPT + kk * WLEN, WLEN)])

        w = _worker(c, s)
        pltpu.sync_copy(s_hbm.at[pl.ds(w * RW, RW)], sidx)
        pltpu.sync_copy(d_hbm.at[pl.ds(w * RW, RW)], didx)
        plsc.subcore_barrier()

        headmaps = [(lax.iota(jnp.int32, 16) + (col0 + 16 * k)) // od
                    for k in range(nk)]

        @pl.loop(0, RW)
        def _(r):
            pltpu.sync_copy(G_hbm.at[sidx.at[r, 0]], g)
            pltpu.sync_copy(er_hbm.at[didx.at[r, 0]], erg)
            if sep_el:
                pltpu.sync_copy(el_hbm.at[sidx.at[r, 0]], elg)

            @pl.loop(0, WLEN // 16)
            def _(ch):
                rows = ch * 16 + lax.iota(jnp.int32, 16)
                for h in range(6):
                    col = jnp.full((16,), h, jnp.int32)
                    if sep_el:
                        elv = plsc.load_gather(elg, [rows, col])
                    else:
                        elv = plsc.load_gather(
                            g, [rows, jnp.full((16,), elcol + h, jnp.int32)])
                    erv = plsc.load_gather(erg, [rows, col])
                    z = elv + erv
                    z = jnp.maximum(z, 0.2 * z)
                    plsc.store_scatter(al, [rows, col], jnp.exp(z))

            @pl.loop(0, WLEN)
            def _(e):
                erow = jnp.full((16,), e, jnp.int32)
                for k in range(nk):
                    av = plsc.load_gather(al, [erow, headmaps[k]])
                    g[e, pl.ds(16 * k, 16)] = g[e, pl.ds(16 * k, 16)] * av

            pltpu.sync_copy(g, acc.at[didx.at[r, 0]], add=True)
            if dacc is not None:
                pltpu.sync_copy(al, dacc.at[didx.at[r, 0]], add=True)

        plsc.subcore_barrier()

        @pl.loop(0, NCH)
        def _(kk):
            sl = pl.ds(s * RPT + kk * WLEN, WLEN)
            pltpu.sync_copy(acc.at[sl], g)
            pltpu.sync_copy(g, o_hbm.at[c, sl])
            if dacc is not None:
                pltpu.sync_copy(dacc.at[sl], al)
                pltpu.sync_copy(al, od_hbm.at[c, sl])

    deco = functools.partial(
        pl.kernel, mesh=_mesh(), compiler_params=_SC_PARAMS,
        out_type=out_type if len(out_type) > 1 else out_type[0],
        scratch_types=scratch)

    if sep_el and with_denom:
        @deco
        def k(G_hbm, er_hbm, s_hbm, d_hbm, el_hbm, o_hbm, od_hbm,
              sidx, didx, g, erg, elg, al, acc, dacc):
            body(G_hbm, er_hbm, s_hbm, d_hbm, el_hbm, o_hbm, od_hbm,
                 sidx, didx, g, erg, elg, al, acc, dacc)
        return k(G, er16, src2d, dst2d, el16)
    elif sep_el:
        @deco
        def k(G_hbm, er_hbm, s_hbm, d_hbm, el_hbm, o_hbm,
              sidx, didx, g, erg, elg, al, acc):
            body(G_hbm, er_hbm, s_hbm, d_hbm, el_hbm, o_hbm, None,
                 sidx, didx, g, erg, elg, al, acc, None)
        return [k(G, er16, src2d, dst2d, el16)]
    else:
        @deco
        def k(G_hbm, er_hbm, s_hbm, d_hbm, o_hbm, od_hbm,
              sidx, didx, g, erg, al, acc, dacc):
            body(G_hbm, er_hbm, s_hbm, d_hbm, None, o_hbm, od_hbm,
                 sidx, didx, g, erg, None, al, acc, dacc)
        return k(G, er16, src2d, dst2d)


def _graph_conv(x, W, b, src2d, dst2d, norm_src, norm_dst):
    h = (x * norm_src) @ W
    h48 = jnp.pad(h, ((0, 0), (0, 12)))
    aggp = _sc_gcn_agg(h48, src2d, dst2d)
    agg = (aggp[0] + aggp[1])[:, :36]
    return agg * norm_dst + b


def _gat_conv(x, W, A_el, A_er, b, src2d, dst2d, out_dim):
    feat = x @ W
    el = feat @ A_el
    er = feat @ A_er
    er16 = jnp.pad(er, ((0, 0), (0, 10)))
    if out_dim == 6:
        G = jnp.concatenate([feat, el, jnp.zeros((N_A, 6), jnp.float32)], 1)
        msgp, denp = _sc_gat_fused(G, er16, src2d, dst2d, od=6)
        msg = (msgp[0] + msgp[1])[:, :36]
    else:
        el16 = jnp.pad(el, ((0, 0), (0, 10)))
        msgA, denp = _sc_gat_fused(feat[:, :96], er16, src2d, dst2d,
                                   el16=el16, od=out_dim, col0=0)
        (msgB,) = _sc_gat_fused(feat[:, 96:], er16, src2d, dst2d,
                                el16=el16, od=out_dim, col0=96,
                                with_denom=False)
        msg = jnp.concatenate([msgA[0] + msgA[1], msgB[0] + msgB[1]], axis=1)
    denom = (denp[0] + denp[1])[:, :6]
    inv = 1.0 / (denom + 1e-9)
    rst = msg * jnp.repeat(inv, out_dim, axis=1)
    return rst + b.reshape(1, 6 * out_dim)


def _head_matrix(a):
    heads, od = a.shape
    idx = jnp.arange(heads * od)
    return jnp.zeros((heads * od, heads), jnp.float32).at[idx, idx // od].set(
        a.reshape(-1))


def kernel(features, gca1_gcn_W, gca1_gcn_b, gca1_gat_W, gca1_gat_al,
           gca1_gat_ar, gca1_gat_b, gca_gcn_W, gca_gcn_b, gca_gat_W,
           gca_gat_al, gca_gat_ar, gca_gat_b, ma_W, ma_al, ma_ar, ma_b,
           dense_W, dense_b, edge_index, num_blocks_Q, num_blocks_L):
    src = edge_index[0].astype(jnp.int32)
    dst = edge_index[1].astype(jnp.int32)
    npad = E_PAD - E
    pad_ids = (N + (jnp.arange(npad, dtype=jnp.int32) % 32)).astype(jnp.int32)
    src2d = jnp.concatenate([src, pad_ids]).reshape(NW * RW, 1, WLEN)
    dst2d = jnp.concatenate([dst, pad_ids]).reshape(NW * RW, 1, WLEN)

    do0, do1, di0, di1 = _sc_degrees(src2d, dst2d)
    deg_out = do0 + do1
    deg_in = di0 + di1
    norm_src = (jnp.where(deg_out > 0, deg_out, 1.0) ** -0.5)[:, None]
    norm_dst = (jnp.where(deg_in > 0, deg_in, 1.0) ** -0.5)[:, None]

    x0 = jnp.pad(features, ((0, N_A - N), (0, 0)))

    A1_el = _head_matrix(gca1_gat_al)
    A1_er = _head_matrix(gca1_gat_ar)
    A_el = _head_matrix(gca_gat_al)
    A_er = _head_matrix(gca_gat_ar)
    Am_el = _head_matrix(ma_al)
    Am_er = _head_matrix(ma_ar)

    def gca1(x):
        h = _graph_conv(x, gca1_gcn_W, gca1_gcn_b, src2d, dst2d,
                        norm_src, norm_dst)
        return _gat_conv(h, gca1_gat_W, A1_el, A1_er, gca1_gat_b,
                         src2d, dst2d, 6)

    def gca(x):
        h = _graph_conv(x, gca_gcn_W, gca_gcn_b, src2d, dst2d,
                        norm_src, norm_dst)
        return _gat_conv(h, gca_gat_W, A_el, A_er, gca_gat_b, src2d, dst2d, 6)

    def _residual_block(i, x):
        return x + gca(gca(x))

    x = gca1(x0)
    x = jax.lax.fori_loop(0, num_blocks_Q // 2, _residual_block, x)
    x = _gat_conv(x, ma_W, Am_el, Am_er, ma_b, src2d, dst2d, 32)
    x = gca1(x)
    x = jax.lax.fori_loop(0, num_blocks_L // 2, _residual_block, x)
    return x[:N] @ dense_W + dense_b


# speedup vs baseline: 82.9269x; 1.1992x over previous
import functools

import jax
import jax.numpy as jnp
from jax import lax
from jax.experimental import pallas as pl
from jax.experimental.pallas import tpu as pltpu
from jax.experimental.pallas import tpu_sc as plsc

N = 10000
E = 320000

NC = 2
NS = 16
NW = NC * NS
WLEN = 128
RW = 80
E_PAD = NW * RW * WLEN
N_A = 10240
RPT = N_A // NS
NCH = RPT // WLEN

_mesh = functools.partial(plsc.VectorSubcoreMesh,
                          core_axis_name="c", subcore_axis_name="s")
_SC_PARAMS = pltpu.CompilerParams(use_tc_tiling_on_sc=False,
                                  needs_layout_passes=False)


def _worker(c, s):
    return c * NS + s


def _sc_degrees(src2d, dst2d):
    @functools.partial(
        pl.kernel, mesh=_mesh(), compiler_params=_SC_PARAMS,
        out_type=[jax.ShapeDtypeStruct((N_A,), jnp.float32),
                  jax.ShapeDtypeStruct((N_A,), jnp.float32),
                  jax.ShapeDtypeStruct((N_A,), jnp.float32),
                  jax.ShapeDtypeStruct((N_A,), jnp.float32)],
        scratch_types=[
            pltpu.VMEM((RW, 1, WLEN), jnp.int32),
            pltpu.VMEM((RW, 1, WLEN), jnp.int32),
            pltpu.VMEM((WLEN,), jnp.float32),
            pltpu.VMEM((WLEN,), jnp.float32),
            pltpu.VMEM_SHARED((N_A,), jnp.float32),
            pltpu.VMEM_SHARED((N_A,), jnp.float32),
        ])
    def k(s_hbm, d_hbm, do0_hbm, do1_hbm, di0_hbm, di1_hbm,
          sidx, didx, ones_v, stg_v, acc_o, acc_i):
        c = lax.axis_index("c")
        s = lax.axis_index("s")

        @pl.loop(0, WLEN // 16)
        def _(i):
            ones_v[pl.ds(i * 16, 16)] = jnp.full((16,), 1.0, jnp.float32)
            stg_v[pl.ds(i * 16, 16)] = jnp.zeros((16,), jnp.float32)

        @pl.loop(0, NCH)
        def _(kk):
            sl = pl.ds(s * RPT + kk * WLEN, WLEN)
            pltpu.sync_copy(stg_v, acc_o.at[sl])
            pltpu.sync_copy(stg_v, acc_i.at[sl])

        w = _worker(c, s)
        pltpu.sync_copy(s_hbm.at[pl.ds(w * RW, RW)], sidx)
        pltpu.sync_copy(d_hbm.at[pl.ds(w * RW, RW)], didx)
        plsc.subcore_barrier()

        @pl.loop(0, RW)
        def _(r):
            pltpu.sync_copy(ones_v, acc_o.at[sidx.at[r, 0]], add=True)
            pltpu.sync_copy(ones_v, acc_i.at[didx.at[r, 0]], add=True)

        plsc.subcore_barrier()

        @pl.loop(0, NCH)
        def _(kk):
            sl = pl.ds(s * RPT + kk * WLEN, WLEN)

            @pl.when(c == 0)
            def _():
                pltpu.sync_copy(acc_o.at[sl], stg_v)
                pltpu.sync_copy(stg_v, do0_hbm.at[sl])
                pltpu.sync_copy(acc_i.at[sl], stg_v)
                pltpu.sync_copy(stg_v, di0_hbm.at[sl])

            @pl.when(c == 1)
            def _():
                pltpu.sync_copy(acc_o.at[sl], stg_v)
                pltpu.sync_copy(stg_v, do1_hbm.at[sl])
                pltpu.sync_copy(acc_i.at[sl], stg_v)
                pltpu.sync_copy(stg_v, di1_hbm.at[sl])

    return k(src2d, dst2d)


def _sc_gcn_agg(h48, src2d, dst2d):
    @functools.partial(
        pl.kernel, mesh=_mesh(), compiler_params=_SC_PARAMS,
        out_type=jax.ShapeDtypeStruct((NC, N_A, 48), jnp.float32),
        scratch_types=[
            pltpu.VMEM((RW, 1, WLEN), jnp.int32),
            pltpu.VMEM((RW, 1, WLEN), jnp.int32),
            pltpu.VMEM((WLEN, 48), jnp.float32),
            pltpu.VMEM((WLEN, 48), jnp.float32),
            pltpu.VMEM_SHARED((N_A, 48), jnp.float32),
            pltpu.SemaphoreType.DMA,
            pltpu.SemaphoreType.DMA,
            pltpu.SemaphoreType.DMA,
            pltpu.SemaphoreType.DMA,
        ])
    def k(h_hbm, s_hbm, d_hbm, o_hbm, sidx, didx, g0, g1, acc,
          gs0, gs1, ss0, ss1):
        c = lax.axis_index("c")
        s = lax.axis_index("s")
        g = (g0, g1)
        gs = (gs0, gs1)
        ss = (ss0, ss1)

        @pl.loop(0, WLEN)
        def _(r):
            for j in range(3):
                g0[r, pl.ds(j * 16, 16)] = jnp.zeros((16,), jnp.float32)

        @pl.loop(0, NCH)
        def _(kk):
            pltpu.sync_copy(g0, acc.at[pl.ds(s * RPT + kk * WLEN, WLEN)])

        w = _worker(c, s)
        pltpu.sync_copy(s_hbm.at[pl.ds(w * RW, RW)], sidx)
        pltpu.sync_copy(d_hbm.at[pl.ds(w * RW, RW)], didx)
        plsc.subcore_barrier()

        def gather(j, r):
            pltpu.async_copy(h_hbm.at[sidx.at[r, 0]], g[j], gs[j])

        def wait_gather(j, r):
            pltpu.make_async_copy(h_hbm.at[sidx.at[r, 0]], g[j], gs[j]).wait()

        def scatter(j, r):
            pltpu.async_copy(g[j], acc.at[didx.at[r, 0]], ss[j], add=True)

        def wait_scatter(j, r):
            pltpu.make_async_copy(g[j], acc.at[didx.at[r, 0]], ss[j]).wait()

        gather(0, 0)

        @pl.loop(0, RW, step=2)
        def _(r):
            for j in (0, 1):
                rr = r + j
                o = 1 - j
                wait_gather(j, rr)
                scatter(j, rr)

                @pl.when(r + j + 1 < RW)
                def _():
                    @pl.when(r + j >= 1)
                    def _():
                        wait_scatter(o, rr - 1)
                    gather(o, rr + 1)

        wait_scatter(0, RW - 2)
        wait_scatter(1, RW - 1)
        plsc.subcore_barrier()

        @pl.loop(0, NCH)
        def _(kk):
            sl = pl.ds(s * RPT + kk * WLEN, WLEN)
            pltpu.sync_copy(acc.at[sl], g0)
            pltpu.sync_copy(g0, o_hbm.at[c, sl])

    return k(h48, src2d, dst2d)


def _sc_gat_fused(G, er16, src2d, dst2d, el16=None, elcol=36, od=6, col0=0,
                  with_denom=True):
    GW = G.shape[1]
    nk = GW // 16
    sep_el = el16 is not None
    scratch = [
        pltpu.VMEM((RW, 1, WLEN), jnp.int32),
        pltpu.VMEM((RW, 1, WLEN), jnp.int32),
        pltpu.VMEM((WLEN, GW), jnp.float32),
        pltpu.VMEM((WLEN, GW), jnp.float32),
        pltpu.VMEM((WLEN, 16), jnp.float32),
        pltpu.VMEM((WLEN, 16), jnp.float32),
        pltpu.VMEM((WLEN, 16), jnp.float32),
        pltpu.VMEM((WLEN, 16), jnp.float32),
        pltpu.VMEM_SHARED((N_A, GW), jnp.float32),
        pltpu.SemaphoreType.DMA,
        pltpu.SemaphoreType.DMA,
        pltpu.SemaphoreType.DMA,
        pltpu.SemaphoreType.DMA,
    ]
    if sep_el:
        scratch.insert(8, pltpu.VMEM((WLEN, 16), jnp.float32))
        scratch.insert(8, pltpu.VMEM((WLEN, 16), jnp.float32))
    out_type = [jax.ShapeDtypeStruct((NC, N_A, GW), jnp.float32)]
    if with_denom:
        scratch.append(pltpu.VMEM_SHARED((N_A, 16), jnp.float32))
        out_type.append(jax.ShapeDtypeStruct((NC, N_A, 16), jnp.float32))

    def body(G_hbm, er_hbm, s_hbm, d_hbm, el_hbm, o_hbm, od_hbm,
             sidx, didx, g, erg, elg, al, acc, dacc, gs, ss):
        c = lax.axis_index("c")
        s = lax.axis_index("s")

        @pl.loop(0, WLEN)
        def _(r):
            for j in range(nk):
                g[0][r, pl.ds(j * 16, 16)] = jnp.zeros((16,), jnp.float32)
            al[0][r, :] = jnp.zeros((16,), jnp.float32)
            al[1][r, :] = jnp.zeros((16,), jnp.float32)

        @pl.loop(0, NCH)
        def _(kk):
            pltpu.sync_copy(g[0], acc.at[pl.ds(s * RPT + kk * WLEN, WLEN)])
            if dacc is not None:
                pltpu.sync_copy(al[0], dacc.at[pl.ds(s * RPT + kk * WLEN, WLEN)])

        w = _worker(c, s)
        pltpu.sync_copy(s_hbm.at[pl.ds(w * RW, RW)], sidx)
        pltpu.sync_copy(d_hbm.at[pl.ds(w * RW, RW)], didx)
        plsc.subcore_barrier()

        headmaps = [(lax.iota(jnp.int32, 16) + (col0 + 16 * k)) // od
                    for k in range(nk)]

        def gather(j, r):
            pltpu.async_copy(G_hbm.at[sidx.at[r, 0]], g[j], gs[j])
            pltpu.async_copy(er_hbm.at[didx.at[r, 0]], erg[j], gs[j])
            if sep_el:
                pltpu.async_copy(el_hbm.at[sidx.at[r, 0]], elg[j], gs[j])

        def wait_gather(j, r):
            pltpu.make_async_copy(G_hbm.at[sidx.at[r, 0]], g[j], gs[j]).wait()
            pltpu.make_async_copy(er_hbm.at[didx.at[r, 0]], erg[j], gs[j]).wait()
            if sep_el:
                pltpu.make_async_copy(el_hbm.at[sidx.at[r, 0]], elg[j], gs[j]).wait()

        def scatter(j, r):
            pltpu.async_copy(g[j], acc.at[didx.at[r, 0]], ss[j], add=True)
            if dacc is not None:
                pltpu.async_copy(al[j], dacc.at[didx.at[r, 0]], ss[j], add=True)

        def wait_scatter(j, r):
            pltpu.make_async_copy(g[j], acc.at[didx.at[r, 0]], ss[j]).wait()
            if dacc is not None:
                pltpu.make_async_copy(al[j], dacc.at[didx.at[r, 0]], ss[j]).wait()

        gather(0, 0)

        @pl.loop(0, RW, step=2)
        def _(r):
            for j in (0, 1):
                rr = r + j
                o = 1 - j
                wait_gather(j, rr)

                @pl.loop(0, WLEN // 16)
                def _(ch):
                    rows = ch * 16 + lax.iota(jnp.int32, 16)
                    for h in range(6):
                        col = jnp.full((16,), h, jnp.int32)
                        if sep_el:
                            elv = plsc.load_gather(elg[j], [rows, col])
                        else:
                            elv = plsc.load_gather(
                                g[j], [rows, jnp.full((16,), elcol + h, jnp.int32)])
                        erv = plsc.load_gather(erg[j], [rows, col])
                        z = elv + erv
                        z = jnp.maximum(z, 0.2 * z)
                        plsc.store_scatter(al[j], [rows, col], jnp.exp(z))

                @pl.loop(0, WLEN)
                def _(e):
                    erow = jnp.full((16,), e, jnp.int32)
                    for k in range(nk):
                        av = plsc.load_gather(al[j], [erow, headmaps[k]])
                        g[j][e, pl.ds(16 * k, 16)] = g[j][e, pl.ds(16 * k, 16)] * av

                scatter(j, rr)

                @pl.when(r + j + 1 < RW)
                def _():
                    @pl.when(r + j >= 1)
                    def _():
                        wait_scatter(o, rr - 1)
                    gather(o, rr + 1)

        wait_scatter(0, RW - 2)
        wait_scatter(1, RW - 1)
        plsc.subcore_barrier()

        @pl.loop(0, NCH)
        def _(kk):
            sl = pl.ds(s * RPT + kk * WLEN, WLEN)
            pltpu.sync_copy(acc.at[sl], g[0])
            pltpu.sync_copy(g[0], o_hbm.at[c, sl])
            if dacc is not None:
                pltpu.sync_copy(dacc.at[sl], al[0])
                pltpu.sync_copy(al[0], od_hbm.at[c, sl])

    deco = functools.partial(
        pl.kernel, mesh=_mesh(), compiler_params=_SC_PARAMS,
        out_type=out_type if len(out_type) > 1 else out_type[0],
        scratch_types=scratch)

    if sep_el and with_denom:
        @deco
        def k(G_hbm, er_hbm, s_hbm, d_hbm, el_hbm, o_hbm, od_hbm,
              sidx, didx, ga, gb, era, erb, ala, alb, ela, elb, acc,
              gs0, gs1, ss0, ss1, dacc):
            body(G_hbm, er_hbm, s_hbm, d_hbm, el_hbm, o_hbm, od_hbm,
                 sidx, didx, (ga, gb), (era, erb), (ela, elb), (ala, alb),
                 acc, dacc, (gs0, gs1), (ss0, ss1))
        return k(G, er16, src2d, dst2d, el16)
    elif sep_el:
        @deco
        def k(G_hbm, er_hbm, s_hbm, d_hbm, el_hbm, o_hbm,
              sidx, didx, ga, gb, era, erb, ala, alb, ela, elb, acc,
              gs0, gs1, ss0, ss1):
            body(G_hbm, er_hbm, s_hbm, d_hbm, el_hbm, o_hbm, None,
                 sidx, didx, (ga, gb), (era, erb), (ela, elb), (ala, alb),
                 acc, None, (gs0, gs1), (ss0, ss1))
        return [k(G, er16, src2d, dst2d, el16)]
    else:
        @deco
        def k(G_hbm, er_hbm, s_hbm, d_hbm, o_hbm, od_hbm,
              sidx, didx, ga, gb, era, erb, ala, alb, acc,
              gs0, gs1, ss0, ss1, dacc):
            body(G_hbm, er_hbm, s_hbm, d_hbm, None, o_hbm, od_hbm,
                 sidx, didx, (ga, gb), (era, erb), None, (ala, alb),
                 acc, dacc, (gs0, gs1), (ss0, ss1))
        return k(G, er16, src2d, dst2d)


def _graph_conv(x, W, b, src2d, dst2d, norm_src, norm_dst):
    h = (x * norm_src) @ W
    h48 = jnp.pad(h, ((0, 0), (0, 12)))
    aggp = _sc_gcn_agg(h48, src2d, dst2d)
    agg = (aggp[0] + aggp[1])[:, :36]
    return agg * norm_dst + b


def _gat_conv(x, W, A_el, A_er, b, src2d, dst2d, out_dim):
    feat = x @ W
    el = feat @ A_el
    er = feat @ A_er
    er16 = jnp.pad(er, ((0, 0), (0, 10)))
    if out_dim == 6:
        G = jnp.concatenate([feat, el, jnp.zeros((N_A, 6), jnp.float32)], 1)
        msgp, denp = _sc_gat_fused(G, er16, src2d, dst2d, od=6)
        msg = (msgp[0] + msgp[1])[:, :36]
    else:
        el16 = jnp.pad(el, ((0, 0), (0, 10)))
        msgA, denp = _sc_gat_fused(feat[:, :96], er16, src2d, dst2d,
                                   el16=el16, od=out_dim, col0=0)
        (msgB,) = _sc_gat_fused(feat[:, 96:], er16, src2d, dst2d,
                                el16=el16, od=out_dim, col0=96,
                                with_denom=False)
        msg = jnp.concatenate([msgA[0] + msgA[1], msgB[0] + msgB[1]], axis=1)
    denom = (denp[0] + denp[1])[:, :6]
    inv = 1.0 / (denom + 1e-9)
    rst = msg * jnp.repeat(inv, out_dim, axis=1)
    return rst + b.reshape(1, 6 * out_dim)


def _head_matrix(a):
    heads, od = a.shape
    idx = jnp.arange(heads * od)
    return jnp.zeros((heads * od, heads), jnp.float32).at[idx, idx // od].set(
        a.reshape(-1))


def kernel(features, gca1_gcn_W, gca1_gcn_b, gca1_gat_W, gca1_gat_al,
           gca1_gat_ar, gca1_gat_b, gca_gcn_W, gca_gcn_b, gca_gat_W,
           gca_gat_al, gca_gat_ar, gca_gat_b, ma_W, ma_al, ma_ar, ma_b,
           dense_W, dense_b, edge_index, num_blocks_Q, num_blocks_L):
    src = edge_index[0].astype(jnp.int32)
    dst = edge_index[1].astype(jnp.int32)
    npad = E_PAD - E
    pad_ids = (N + (jnp.arange(npad, dtype=jnp.int32) % 32)).astype(jnp.int32)
    src2d = jnp.concatenate([src, pad_ids]).reshape(NW * RW, 1, WLEN)
    dst2d = jnp.concatenate([dst, pad_ids]).reshape(NW * RW, 1, WLEN)

    do0, do1, di0, di1 = _sc_degrees(src2d, dst2d)
    deg_out = do0 + do1
    deg_in = di0 + di1
    norm_src = (jnp.where(deg_out > 0, deg_out, 1.0) ** -0.5)[:, None]
    norm_dst = (jnp.where(deg_in > 0, deg_in, 1.0) ** -0.5)[:, None]

    x0 = jnp.pad(features, ((0, N_A - N), (0, 0)))

    A1_el = _head_matrix(gca1_gat_al)
    A1_er = _head_matrix(gca1_gat_ar)
    A_el = _head_matrix(gca_gat_al)
    A_er = _head_matrix(gca_gat_ar)
    Am_el = _head_matrix(ma_al)
    Am_er = _head_matrix(ma_ar)

    def gca1(x):
        h = _graph_conv(x, gca1_gcn_W, gca1_gcn_b, src2d, dst2d,
                        norm_src, norm_dst)
        return _gat_conv(h, gca1_gat_W, A1_el, A1_er, gca1_gat_b,
                         src2d, dst2d, 6)

    def gca(x):
        h = _graph_conv(x, gca_gcn_W, gca_gcn_b, src2d, dst2d,
                        norm_src, norm_dst)
        return _gat_conv(h, gca_gat_W, A_el, A_er, gca_gat_b, src2d, dst2d, 6)

    def _residual_block(i, x):
        return x + gca(gca(x))

    x = gca1(x0)
    x = jax.lax.fori_loop(0, num_blocks_Q // 2, _residual_block, x)
    x = _gat_conv(x, ma_W, Am_el, Am_er, ma_b, src2d, dst2d, 32)
    x = gca1(x)
    x = jax.lax.fori_loop(0, num_blocks_L // 2, _residual_block, x)
    return x[:N] @ dense_W + dense_b


# speedup vs baseline: 83.5783x; 1.0079x over previous
import functools

import jax
import jax.numpy as jnp
from jax import lax
from jax.experimental import pallas as pl
from jax.experimental.pallas import tpu as pltpu
from jax.experimental.pallas import tpu_sc as plsc

N = 10000
E = 320000

NC = 2
NS = 16
NW = NC * NS
WLEN = 128
RW = 80
E_PAD = NW * RW * WLEN
N_A = 10240
RPT = N_A // NS
NCH = RPT // WLEN

_mesh = functools.partial(plsc.VectorSubcoreMesh,
                          core_axis_name="c", subcore_axis_name="s")
_SC_PARAMS = pltpu.CompilerParams(use_tc_tiling_on_sc=False,
                                  needs_layout_passes=False)


def _worker(c, s):
    return c * NS + s


def _sc_degrees(src2d, dst2d):
    @functools.partial(
        pl.kernel, mesh=_mesh(), compiler_params=_SC_PARAMS,
        out_type=[jax.ShapeDtypeStruct((N_A,), jnp.float32),
                  jax.ShapeDtypeStruct((N_A,), jnp.float32),
                  jax.ShapeDtypeStruct((N_A,), jnp.float32),
                  jax.ShapeDtypeStruct((N_A,), jnp.float32)],
        scratch_types=[
            pltpu.VMEM((RW, 1, WLEN), jnp.int32),
            pltpu.VMEM((RW, 1, WLEN), jnp.int32),
            pltpu.VMEM((WLEN,), jnp.float32),
            pltpu.VMEM((WLEN,), jnp.float32),
            pltpu.VMEM_SHARED((N_A,), jnp.float32),
            pltpu.VMEM_SHARED((N_A,), jnp.float32),
        ])
    def k(s_hbm, d_hbm, do0_hbm, do1_hbm, di0_hbm, di1_hbm,
          sidx, didx, ones_v, stg_v, acc_o, acc_i):
        c = lax.axis_index("c")
        s = lax.axis_index("s")

        @pl.loop(0, WLEN // 16)
        def _(i):
            ones_v[pl.ds(i * 16, 16)] = jnp.full((16,), 1.0, jnp.float32)
            stg_v[pl.ds(i * 16, 16)] = jnp.zeros((16,), jnp.float32)

        @pl.loop(0, NCH)
        def _(kk):
            sl = pl.ds(s * RPT + kk * WLEN, WLEN)
            pltpu.sync_copy(stg_v, acc_o.at[sl])
            pltpu.sync_copy(stg_v, acc_i.at[sl])

        w = _worker(c, s)
        pltpu.sync_copy(s_hbm.at[pl.ds(w * RW, RW)], sidx)
        pltpu.sync_copy(d_hbm.at[pl.ds(w * RW, RW)], didx)
        plsc.subcore_barrier()

        @pl.loop(0, RW)
        def _(r):
            pltpu.sync_copy(ones_v, acc_o.at[sidx.at[r, 0]], add=True)
            pltpu.sync_copy(ones_v, acc_i.at[didx.at[r, 0]], add=True)

        plsc.subcore_barrier()

        @pl.loop(0, NCH)
        def _(kk):
            sl = pl.ds(s * RPT + kk * WLEN, WLEN)

            @pl.when(c == 0)
            def _():
                pltpu.sync_copy(acc_o.at[sl], stg_v)
                pltpu.sync_copy(stg_v, do0_hbm.at[sl])
                pltpu.sync_copy(acc_i.at[sl], stg_v)
                pltpu.sync_copy(stg_v, di0_hbm.at[sl])

            @pl.when(c == 1)
            def _():
                pltpu.sync_copy(acc_o.at[sl], stg_v)
                pltpu.sync_copy(stg_v, do1_hbm.at[sl])
                pltpu.sync_copy(acc_i.at[sl], stg_v)
                pltpu.sync_copy(stg_v, di1_hbm.at[sl])

    return k(src2d, dst2d)


def _sc_gcn_agg(h48, src2d, dst2d):
    @functools.partial(
        pl.kernel, mesh=_mesh(), compiler_params=_SC_PARAMS,
        out_type=jax.ShapeDtypeStruct((NC, N_A, 48), jnp.float32),
        scratch_types=[
            pltpu.VMEM((RW, 1, WLEN), jnp.int32),
            pltpu.VMEM((RW, 1, WLEN), jnp.int32),
            pltpu.VMEM((WLEN, 48), jnp.float32),
            pltpu.VMEM((WLEN, 48), jnp.float32),
            pltpu.VMEM_SHARED((N_A, 48), jnp.float32),
            pltpu.SemaphoreType.DMA,
            pltpu.SemaphoreType.DMA,
            pltpu.SemaphoreType.DMA,
            pltpu.SemaphoreType.DMA,
        ])
    def k(h_hbm, s_hbm, d_hbm, o_hbm, sidx, didx, g0, g1, acc,
          gs0, gs1, ss0, ss1):
        c = lax.axis_index("c")
        s = lax.axis_index("s")
        g = (g0, g1)
        gs = (gs0, gs1)
        ss = (ss0, ss1)

        @pl.loop(0, WLEN)
        def _(r):
            for j in range(3):
                g0[r, pl.ds(j * 16, 16)] = jnp.zeros((16,), jnp.float32)

        @pl.loop(0, NCH)
        def _(kk):
            pltpu.sync_copy(g0, acc.at[pl.ds(s * RPT + kk * WLEN, WLEN)])

        w = _worker(c, s)
        pltpu.sync_copy(s_hbm.at[pl.ds(w * RW, RW)], sidx)
        pltpu.sync_copy(d_hbm.at[pl.ds(w * RW, RW)], didx)
        plsc.subcore_barrier()

        def gather(j, r):
            pltpu.async_copy(h_hbm.at[sidx.at[r, 0]], g[j], gs[j])

        def wait_gather(j, r):
            pltpu.make_async_copy(h_hbm.at[sidx.at[r, 0]], g[j], gs[j]).wait()

        def scatter(j, r):
            pltpu.async_copy(g[j], acc.at[didx.at[r, 0]], ss[j], add=True)

        def wait_scatter(j, r):
            pltpu.make_async_copy(g[j], acc.at[didx.at[r, 0]], ss[j]).wait()

        gather(0, 0)

        @pl.loop(0, RW, step=2)
        def _(r):
            for j in (0, 1):
                rr = r + j
                o = 1 - j
                wait_gather(j, rr)
                scatter(j, rr)

                @pl.when(r + j + 1 < RW)
                def _():
                    @pl.when(r + j >= 1)
                    def _():
                        wait_scatter(o, rr - 1)
                    gather(o, rr + 1)

        wait_scatter(0, RW - 2)
        wait_scatter(1, RW - 1)
        plsc.subcore_barrier()

        @pl.loop(0, NCH)
        def _(kk):
            sl = pl.ds(s * RPT + kk * WLEN, WLEN)
            pltpu.sync_copy(acc.at[sl], g0)
            pltpu.sync_copy(g0, o_hbm.at[c, sl])

    return k(h48, src2d, dst2d)


def _sc_gat_fused(G, er16, src2d, dst2d, el16=None, elcol=36, od=6, col0=0,
                  with_denom=True):
    GW = G.shape[1]
    nk = GW // 16
    sep_el = el16 is not None
    scratch = [
        pltpu.VMEM((RW, 1, WLEN), jnp.int32),
        pltpu.VMEM((RW, 1, WLEN), jnp.int32),
        pltpu.VMEM((WLEN, GW), jnp.float32),
        pltpu.VMEM((WLEN, GW), jnp.float32),
        pltpu.VMEM((WLEN, 16), jnp.float32),
        pltpu.VMEM((WLEN, 16), jnp.float32),
        pltpu.VMEM((WLEN, 16), jnp.float32),
        pltpu.VMEM((WLEN, 16), jnp.float32),
        pltpu.VMEM_SHARED((N_A, GW), jnp.float32),
        pltpu.SemaphoreType.DMA,
        pltpu.SemaphoreType.DMA,
        pltpu.SemaphoreType.DMA,
        pltpu.SemaphoreType.DMA,
    ]
    if sep_el:
        scratch.insert(8, pltpu.VMEM((WLEN, 16), jnp.float32))
        scratch.insert(8, pltpu.VMEM((WLEN, 16), jnp.float32))
    out_type = [jax.ShapeDtypeStruct((NC, N_A, GW), jnp.float32)]
    if with_denom:
        scratch.append(pltpu.VMEM_SHARED((N_A, 16), jnp.float32))
        out_type.append(jax.ShapeDtypeStruct((NC, N_A, 16), jnp.float32))

    def body(G_hbm, er_hbm, s_hbm, d_hbm, el_hbm, o_hbm, od_hbm,
             sidx, didx, g, erg, elg, al, acc, dacc, gs, ss):
        c = lax.axis_index("c")
        s = lax.axis_index("s")

        @pl.loop(0, WLEN)
        def _(r):
            for j in range(nk):
                g[0][r, pl.ds(j * 16, 16)] = jnp.zeros((16,), jnp.float32)
            al[0][r, :] = jnp.zeros((16,), jnp.float32)
            al[1][r, :] = jnp.zeros((16,), jnp.float32)

        @pl.loop(0, NCH)
        def _(kk):
            pltpu.sync_copy(g[0], acc.at[pl.ds(s * RPT + kk * WLEN, WLEN)])
            if dacc is not None:
                pltpu.sync_copy(al[0], dacc.at[pl.ds(s * RPT + kk * WLEN, WLEN)])

        w = _worker(c, s)
        pltpu.sync_copy(s_hbm.at[pl.ds(w * RW, RW)], sidx)
        pltpu.sync_copy(d_hbm.at[pl.ds(w * RW, RW)], didx)
        plsc.subcore_barrier()

        headmaps = [(lax.iota(jnp.int32, 16) + (col0 + 16 * k)) // od
                    for k in range(nk)]

        def gather(j, r):
            pltpu.async_copy(G_hbm.at[sidx.at[r, 0]], g[j], gs[j])
            pltpu.async_copy(er_hbm.at[didx.at[r, 0]], erg[j], gs[j])
            if sep_el:
                pltpu.async_copy(el_hbm.at[sidx.at[r, 0]], elg[j], gs[j])

        def wait_gather(j, r):
            pltpu.make_async_copy(G_hbm.at[sidx.at[r, 0]], g[j], gs[j]).wait()
            pltpu.make_async_copy(er_hbm.at[didx.at[r, 0]], erg[j], gs[j]).wait()
            if sep_el:
                pltpu.make_async_copy(el_hbm.at[sidx.at[r, 0]], elg[j], gs[j]).wait()

        def scatter(j, r):
            pltpu.async_copy(g[j], acc.at[didx.at[r, 0]], ss[j], add=True)
            if dacc is not None:
                pltpu.async_copy(al[j], dacc.at[didx.at[r, 0]], ss[j], add=True)

        def wait_scatter(j, r):
            pltpu.make_async_copy(g[j], acc.at[didx.at[r, 0]], ss[j]).wait()
            if dacc is not None:
                pltpu.make_async_copy(al[j], dacc.at[didx.at[r, 0]], ss[j]).wait()

        gather(0, 0)

        @pl.loop(0, RW, step=2)
        def _(r):
            for j in (0, 1):
                rr = r + j
                o = 1 - j
                wait_gather(j, rr)

                @pl.loop(0, WLEN // 16, step=2)
                def _(cb):
                    for cj in range(2):
                        rows = (cb + cj) * 16 + lax.iota(jnp.int32, 16)
                        for h in range(6):
                            col = jnp.full((16,), h, jnp.int32)
                            if sep_el:
                                elv = plsc.load_gather(elg[j], [rows, col])
                            else:
                                elv = plsc.load_gather(
                                    g[j],
                                    [rows, jnp.full((16,), elcol + h, jnp.int32)])
                            erv = plsc.load_gather(erg[j], [rows, col])
                            z = elv + erv
                            z = jnp.maximum(z, 0.2 * z)
                            plsc.store_scatter(al[j], [rows, col], jnp.exp(z))

                @pl.loop(0, WLEN, step=4)
                def _(eb):
                    for jj in range(4):
                        e = eb + jj
                        erow = jnp.full((16,), e, jnp.int32)
                        for k in range(nk):
                            av = plsc.load_gather(al[j], [erow, headmaps[k]])
                            g[j][e, pl.ds(16 * k, 16)] = (
                                g[j][e, pl.ds(16 * k, 16)] * av)

                scatter(j, rr)

                @pl.when(r + j + 1 < RW)
                def _():
                    @pl.when(r + j >= 1)
                    def _():
                        wait_scatter(o, rr - 1)
                    gather(o, rr + 1)

        wait_scatter(0, RW - 2)
        wait_scatter(1, RW - 1)
        plsc.subcore_barrier()

        @pl.loop(0, NCH)
        def _(kk):
            sl = pl.ds(s * RPT + kk * WLEN, WLEN)
            pltpu.sync_copy(acc.at[sl], g[0])
            pltpu.sync_copy(g[0], o_hbm.at[c, sl])
            if dacc is not None:
                pltpu.sync_copy(dacc.at[sl], al[0])
                pltpu.sync_copy(al[0], od_hbm.at[c, sl])

    deco = functools.partial(
        pl.kernel, mesh=_mesh(), compiler_params=_SC_PARAMS,
        out_type=out_type if len(out_type) > 1 else out_type[0],
        scratch_types=scratch)

    if sep_el and with_denom:
        @deco
        def k(G_hbm, er_hbm, s_hbm, d_hbm, el_hbm, o_hbm, od_hbm,
              sidx, didx, ga, gb, era, erb, ala, alb, ela, elb, acc,
              gs0, gs1, ss0, ss1, dacc):
            body(G_hbm, er_hbm, s_hbm, d_hbm, el_hbm, o_hbm, od_hbm,
                 sidx, didx, (ga, gb), (era, erb), (ela, elb), (ala, alb),
                 acc, dacc, (gs0, gs1), (ss0, ss1))
        return k(G, er16, src2d, dst2d, el16)
    elif sep_el:
        @deco
        def k(G_hbm, er_hbm, s_hbm, d_hbm, el_hbm, o_hbm,
              sidx, didx, ga, gb, era, erb, ala, alb, ela, elb, acc,
              gs0, gs1, ss0, ss1):
            body(G_hbm, er_hbm, s_hbm, d_hbm, el_hbm, o_hbm, None,
                 sidx, didx, (ga, gb), (era, erb), (ela, elb), (ala, alb),
                 acc, None, (gs0, gs1), (ss0, ss1))
        return [k(G, er16, src2d, dst2d, el16)]
    else:
        @deco
        def k(G_hbm, er_hbm, s_hbm, d_hbm, o_hbm, od_hbm,
              sidx, didx, ga, gb, era, erb, ala, alb, acc,
              gs0, gs1, ss0, ss1, dacc):
            body(G_hbm, er_hbm, s_hbm, d_hbm, None, o_hbm, od_hbm,
                 sidx, didx, (ga, gb), (era, erb), None, (ala, alb),
                 acc, dacc, (gs0, gs1), (ss0, ss1))
        return k(G, er16, src2d, dst2d)


def _graph_conv(x, W, b, src2d, dst2d, norm_src, norm_dst):
    h = (x * norm_src) @ W
    h48 = jnp.pad(h, ((0, 0), (0, 12)))
    aggp = _sc_gcn_agg(h48, src2d, dst2d)
    agg = (aggp[0] + aggp[1])[:, :36]
    return agg * norm_dst + b


def _gat_conv(x, W, A_el, A_er, b, src2d, dst2d, out_dim):
    feat = x @ W
    el = feat @ A_el
    er = feat @ A_er
    er16 = jnp.pad(er, ((0, 0), (0, 10)))
    if out_dim == 6:
        G = jnp.concatenate([feat, el, jnp.zeros((N_A, 6), jnp.float32)], 1)
        msgp, denp = _sc_gat_fused(G, er16, src2d, dst2d, od=6)
        msg = (msgp[0] + msgp[1])[:, :36]
    else:
        el16 = jnp.pad(el, ((0, 0), (0, 10)))
        msgA, denp = _sc_gat_fused(feat[:, :96], er16, src2d, dst2d,
                                   el16=el16, od=out_dim, col0=0)
        (msgB,) = _sc_gat_fused(feat[:, 96:], er16, src2d, dst2d,
                                el16=el16, od=out_dim, col0=96,
                                with_denom=False)
        msg = jnp.concatenate([msgA[0] + msgA[1], msgB[0] + msgB[1]], axis=1)
    denom = (denp[0] + denp[1])[:, :6]
    inv = 1.0 / (denom + 1e-9)
    rst = msg * jnp.repeat(inv, out_dim, axis=1)
    return rst + b.reshape(1, 6 * out_dim)


def _head_matrix(a):
    heads, od = a.shape
    idx = jnp.arange(heads * od)
    return jnp.zeros((heads * od, heads), jnp.float32).at[idx, idx // od].set(
        a.reshape(-1))


def kernel(features, gca1_gcn_W, gca1_gcn_b, gca1_gat_W, gca1_gat_al,
           gca1_gat_ar, gca1_gat_b, gca_gcn_W, gca_gcn_b, gca_gat_W,
           gca_gat_al, gca_gat_ar, gca_gat_b, ma_W, ma_al, ma_ar, ma_b,
           dense_W, dense_b, edge_index, num_blocks_Q, num_blocks_L):
    src = edge_index[0].astype(jnp.int32)
    dst = edge_index[1].astype(jnp.int32)
    npad = E_PAD - E
    pad_ids = (N + (jnp.arange(npad, dtype=jnp.int32) % 32)).astype(jnp.int32)
    src2d = jnp.concatenate([src, pad_ids]).reshape(NW * RW, 1, WLEN)
    dst2d = jnp.concatenate([dst, pad_ids]).reshape(NW * RW, 1, WLEN)

    do0, do1, di0, di1 = _sc_degrees(src2d, dst2d)
    deg_out = do0 + do1
    deg_in = di0 + di1
    norm_src = (jnp.where(deg_out > 0, deg_out, 1.0) ** -0.5)[:, None]
    norm_dst = (jnp.where(deg_in > 0, deg_in, 1.0) ** -0.5)[:, None]

    x0 = jnp.pad(features, ((0, N_A - N), (0, 0)))

    A1_el = _head_matrix(gca1_gat_al)
    A1_er = _head_matrix(gca1_gat_ar)
    A_el = _head_matrix(gca_gat_al)
    A_er = _head_matrix(gca_gat_ar)
    Am_el = _head_matrix(ma_al)
    Am_er = _head_matrix(ma_ar)

    def gca1(x):
        h = _graph_conv(x, gca1_gcn_W, gca1_gcn_b, src2d, dst2d,
                        norm_src, norm_dst)
        return _gat_conv(h, gca1_gat_W, A1_el, A1_er, gca1_gat_b,
                         src2d, dst2d, 6)

    def gca(x):
        h = _graph_conv(x, gca_gcn_W, gca_gcn_b, src2d, dst2d,
                        norm_src, norm_dst)
        return _gat_conv(h, gca_gat_W, A_el, A_er, gca_gat_b, src2d, dst2d, 6)

    def _residual_block(i, x):
        return x + gca(gca(x))

    x = gca1(x0)
    x = jax.lax.fori_loop(0, num_blocks_Q // 2, _residual_block, x)
    x = _gat_conv(x, ma_W, Am_el, Am_er, ma_b, src2d, dst2d, 32)
    x = gca1(x)
    x = jax.lax.fori_loop(0, num_blocks_L // 2, _residual_block, x)
    return x[:N] @ dense_W + dense_b


# speedup vs baseline: 87.0306x; 1.0413x over previous
import functools

import jax
import jax.numpy as jnp
from jax import lax
from jax.experimental import pallas as pl
from jax.experimental.pallas import tpu as pltpu
from jax.experimental.pallas import tpu_sc as plsc

N = 10000
E = 320000

NC = 2
NS = 16
NW = NC * NS
WLEN = 128
RW = 80
E_PAD = NW * RW * WLEN
N_A = 10240
RPT = N_A // NS
NCH = RPT // WLEN

_mesh = functools.partial(plsc.VectorSubcoreMesh,
                          core_axis_name="c", subcore_axis_name="s")
_SC_PARAMS = pltpu.CompilerParams(use_tc_tiling_on_sc=False,
                                  needs_layout_passes=False)


def _worker(c, s):
    return c * NS + s


def _sc_degrees(src2d, dst2d):
    @functools.partial(
        pl.kernel, mesh=_mesh(), compiler_params=_SC_PARAMS,
        out_type=[jax.ShapeDtypeStruct((N_A,), jnp.float32),
                  jax.ShapeDtypeStruct((N_A,), jnp.float32),
                  jax.ShapeDtypeStruct((N_A,), jnp.float32),
                  jax.ShapeDtypeStruct((N_A,), jnp.float32)],
        scratch_types=[
            pltpu.VMEM((RW, 1, WLEN), jnp.int32),
            pltpu.VMEM((RW, 1, WLEN), jnp.int32),
            pltpu.VMEM((WLEN,), jnp.float32),
            pltpu.VMEM((WLEN,), jnp.float32),
            pltpu.VMEM_SHARED((N_A,), jnp.float32),
            pltpu.VMEM_SHARED((N_A,), jnp.float32),
        ])
    def k(s_hbm, d_hbm, do0_hbm, do1_hbm, di0_hbm, di1_hbm,
          sidx, didx, ones_v, stg_v, acc_o, acc_i):
        c = lax.axis_index("c")
        s = lax.axis_index("s")

        @pl.loop(0, WLEN // 16)
        def _(i):
            ones_v[pl.ds(i * 16, 16)] = jnp.full((16,), 1.0, jnp.float32)
            stg_v[pl.ds(i * 16, 16)] = jnp.zeros((16,), jnp.float32)

        @pl.loop(0, NCH)
        def _(kk):
            sl = pl.ds(s * RPT + kk * WLEN, WLEN)
            pltpu.sync_copy(stg_v, acc_o.at[sl])
            pltpu.sync_copy(stg_v, acc_i.at[sl])

        w = _worker(c, s)
        pltpu.sync_copy(s_hbm.at[pl.ds(w * RW, RW)], sidx)
        pltpu.sync_copy(d_hbm.at[pl.ds(w * RW, RW)], didx)
        plsc.subcore_barrier()

        @pl.loop(0, RW)
        def _(r):
            pltpu.sync_copy(ones_v, acc_o.at[sidx.at[r, 0]], add=True)
            pltpu.sync_copy(ones_v, acc_i.at[didx.at[r, 0]], add=True)

        plsc.subcore_barrier()

        @pl.loop(0, NCH)
        def _(kk):
            sl = pl.ds(s * RPT + kk * WLEN, WLEN)

            @pl.when(c == 0)
            def _():
                pltpu.sync_copy(acc_o.at[sl], stg_v)
                pltpu.sync_copy(stg_v, do0_hbm.at[sl])
                pltpu.sync_copy(acc_i.at[sl], stg_v)
                pltpu.sync_copy(stg_v, di0_hbm.at[sl])

            @pl.when(c == 1)
            def _():
                pltpu.sync_copy(acc_o.at[sl], stg_v)
                pltpu.sync_copy(stg_v, do1_hbm.at[sl])
                pltpu.sync_copy(acc_i.at[sl], stg_v)
                pltpu.sync_copy(stg_v, di1_hbm.at[sl])

    return k(src2d, dst2d)


def _sc_gcn_agg(h48, src2d, dst2d):
    @functools.partial(
        pl.kernel, mesh=_mesh(), compiler_params=_SC_PARAMS,
        out_type=jax.ShapeDtypeStruct((NC, N_A, 48), jnp.float32),
        scratch_types=[
            pltpu.VMEM((RW, 1, WLEN), jnp.int32),
            pltpu.VMEM((RW, 1, WLEN), jnp.int32),
            pltpu.VMEM((WLEN, 48), jnp.float32),
            pltpu.VMEM((WLEN, 48), jnp.float32),
            pltpu.VMEM_SHARED((N_A, 48), jnp.float32),
            pltpu.SemaphoreType.DMA,
            pltpu.SemaphoreType.DMA,
            pltpu.SemaphoreType.DMA,
            pltpu.SemaphoreType.DMA,
        ])
    def k(h_hbm, s_hbm, d_hbm, o_hbm, sidx, didx, g0, g1, acc,
          gs0, gs1, ss0, ss1):
        c = lax.axis_index("c")
        s = lax.axis_index("s")
        g = (g0, g1)
        gs = (gs0, gs1)
        ss = (ss0, ss1)

        @pl.loop(0, WLEN)
        def _(r):
            for j in range(3):
                g0[r, pl.ds(j * 16, 16)] = jnp.zeros((16,), jnp.float32)

        @pl.loop(0, NCH)
        def _(kk):
            pltpu.sync_copy(g0, acc.at[pl.ds(s * RPT + kk * WLEN, WLEN)])

        w = _worker(c, s)
        pltpu.sync_copy(s_hbm.at[pl.ds(w * RW, RW)], sidx)
        pltpu.sync_copy(d_hbm.at[pl.ds(w * RW, RW)], didx)
        plsc.subcore_barrier()

        def gather(j, r):
            pltpu.async_copy(h_hbm.at[sidx.at[r, 0]], g[j], gs[j])

        def wait_gather(j, r):
            pltpu.make_async_copy(h_hbm.at[sidx.at[r, 0]], g[j], gs[j]).wait()

        def scatter(j, r):
            pltpu.async_copy(g[j], acc.at[didx.at[r, 0]], ss[j], add=True)

        def wait_scatter(j, r):
            pltpu.make_async_copy(g[j], acc.at[didx.at[r, 0]], ss[j]).wait()

        gather(0, 0)

        @pl.loop(0, RW, step=2)
        def _(r):
            for j in (0, 1):
                rr = r + j
                o = 1 - j
                wait_gather(j, rr)
                scatter(j, rr)

                @pl.when(r + j + 1 < RW)
                def _():
                    @pl.when(r + j >= 1)
                    def _():
                        wait_scatter(o, rr - 1)
                    gather(o, rr + 1)

        wait_scatter(0, RW - 2)
        wait_scatter(1, RW - 1)
        plsc.subcore_barrier()

        @pl.loop(0, NCH)
        def _(kk):
            sl = pl.ds(s * RPT + kk * WLEN, WLEN)
            pltpu.sync_copy(acc.at[sl], g0)
            pltpu.sync_copy(g0, o_hbm.at[c, sl])

    return k(h48, src2d, dst2d)


def _sc_gat_fused(G, er16, src2d, dst2d, el16=None, elcol=36, od=6, col0=0,
                  with_denom=True):
    GW = G.shape[1]
    nk = GW // 16
    sep_el = el16 is not None
    scratch = [
        pltpu.VMEM((RW, 1, WLEN), jnp.int32),
        pltpu.VMEM((RW, 1, WLEN), jnp.int32),
        pltpu.VMEM((WLEN, GW), jnp.float32),
        pltpu.VMEM((WLEN, GW), jnp.float32),
        pltpu.VMEM((WLEN, 16), jnp.float32),
        pltpu.VMEM((WLEN, 16), jnp.float32),
        pltpu.VMEM((WLEN, 16), jnp.float32),
        pltpu.VMEM((WLEN, 16), jnp.float32),
        pltpu.VMEM_SHARED((N_A, GW), jnp.float32),
        pltpu.SemaphoreType.DMA,
        pltpu.SemaphoreType.DMA,
        pltpu.SemaphoreType.DMA,
        pltpu.SemaphoreType.DMA,
    ]
    if sep_el:
        scratch.insert(8, pltpu.VMEM((WLEN, 16), jnp.float32))
        scratch.insert(8, pltpu.VMEM((WLEN, 16), jnp.float32))
    out_type = [jax.ShapeDtypeStruct((NC, N_A, GW), jnp.float32)]
    if with_denom:
        scratch.append(pltpu.VMEM_SHARED((N_A, 16), jnp.float32))
        out_type.append(jax.ShapeDtypeStruct((NC, N_A, 16), jnp.float32))

    def body(G_hbm, er_hbm, s_hbm, d_hbm, el_hbm, o_hbm, od_hbm,
             sidx, didx, g, erg, elg, al, acc, dacc, gs, ss):
        c = lax.axis_index("c")
        s = lax.axis_index("s")

        @pl.loop(0, WLEN)
        def _(r):
            for j in range(nk):
                g[0][r, pl.ds(j * 16, 16)] = jnp.zeros((16,), jnp.float32)
            al[0][r, :] = jnp.zeros((16,), jnp.float32)
            al[1][r, :] = jnp.zeros((16,), jnp.float32)

        @pl.loop(0, NCH)
        def _(kk):
            pltpu.sync_copy(g[0], acc.at[pl.ds(s * RPT + kk * WLEN, WLEN)])
            if dacc is not None:
                pltpu.sync_copy(al[0], dacc.at[pl.ds(s * RPT + kk * WLEN, WLEN)])

        w = _worker(c, s)
        pltpu.sync_copy(s_hbm.at[pl.ds(w * RW, RW)], sidx)
        pltpu.sync_copy(d_hbm.at[pl.ds(w * RW, RW)], didx)
        plsc.subcore_barrier()

        headmaps = [(lax.iota(jnp.int32, 16) + (col0 + 16 * k)) // od
                    for k in range(nk)]

        def gather(j, r):
            pltpu.async_copy(G_hbm.at[sidx.at[r, 0]], g[j], gs[j])
            pltpu.async_copy(er_hbm.at[didx.at[r, 0]], erg[j], gs[j])
            if sep_el:
                pltpu.async_copy(el_hbm.at[sidx.at[r, 0]], elg[j], gs[j])

        def wait_gather(j, r):
            pltpu.make_async_copy(G_hbm.at[sidx.at[r, 0]], g[j], gs[j]).wait()
            pltpu.make_async_copy(er_hbm.at[didx.at[r, 0]], erg[j], gs[j]).wait()
            if sep_el:
                pltpu.make_async_copy(el_hbm.at[sidx.at[r, 0]], elg[j], gs[j]).wait()

        def scatter(j, r):
            pltpu.async_copy(g[j], acc.at[didx.at[r, 0]], ss[j], add=True)
            if dacc is not None:
                pltpu.async_copy(al[j], dacc.at[didx.at[r, 0]], ss[j], add=True)

        def wait_scatter(j, r):
            pltpu.make_async_copy(g[j], acc.at[didx.at[r, 0]], ss[j]).wait()
            if dacc is not None:
                pltpu.make_async_copy(al[j], dacc.at[didx.at[r, 0]], ss[j]).wait()

        gather(0, 0)

        @pl.loop(0, RW, step=2)
        def _(r):
            for j in (0, 1):
                rr = r + j
                o = 1 - j
                wait_gather(j, rr)

                @pl.loop(0, WLEN // 16, step=2)
                def _(cb):
                    for cj in range(2):
                        rows = (cb + cj) * 16 + lax.iota(jnp.int32, 16)
                        for h in range(6):
                            col = jnp.full((16,), h, jnp.int32)
                            if sep_el:
                                elv = plsc.load_gather(elg[j], [rows, col])
                            else:
                                elv = plsc.load_gather(
                                    g[j],
                                    [rows, jnp.full((16,), elcol + h, jnp.int32)])
                            erv = plsc.load_gather(erg[j], [rows, col])
                            z = elv + erv
                            z = jnp.maximum(z, 0.2 * z)
                            plsc.store_scatter(al[j], [rows, col], jnp.exp(z))

                @pl.loop(0, WLEN, step=4)
                def _(eb):
                    for jj in range(4):
                        e = eb + jj
                        erow = jnp.full((16,), e, jnp.int32)
                        for k in range(nk):
                            av = plsc.load_gather(al[j], [erow, headmaps[k]])
                            g[j][e, pl.ds(16 * k, 16)] = (
                                g[j][e, pl.ds(16 * k, 16)] * av)

                scatter(j, rr)

                @pl.when(r + j + 1 < RW)
                def _():
                    @pl.when(r + j >= 1)
                    def _():
                        wait_scatter(o, rr - 1)
                    gather(o, rr + 1)

        wait_scatter(0, RW - 2)
        wait_scatter(1, RW - 1)
        plsc.subcore_barrier()

        @pl.loop(0, NCH)
        def _(kk):
            sl = pl.ds(s * RPT + kk * WLEN, WLEN)
            pltpu.sync_copy(acc.at[sl], g[0])
            pltpu.sync_copy(g[0], o_hbm.at[c, sl])
            if dacc is not None:
                pltpu.sync_copy(dacc.at[sl], al[0])
                pltpu.sync_copy(al[0], od_hbm.at[c, sl])

    deco = functools.partial(
        pl.kernel, mesh=_mesh(), compiler_params=_SC_PARAMS,
        out_type=out_type if len(out_type) > 1 else out_type[0],
        scratch_types=scratch)

    if sep_el and with_denom:
        @deco
        def k(G_hbm, er_hbm, s_hbm, d_hbm, el_hbm, o_hbm, od_hbm,
              sidx, didx, ga, gb, era, erb, ala, alb, ela, elb, acc,
              gs0, gs1, ss0, ss1, dacc):
            body(G_hbm, er_hbm, s_hbm, d_hbm, el_hbm, o_hbm, od_hbm,
                 sidx, didx, (ga, gb), (era, erb), (ela, elb), (ala, alb),
                 acc, dacc, (gs0, gs1), (ss0, ss1))
        return k(G, er16, src2d, dst2d, el16)
    elif sep_el:
        @deco
        def k(G_hbm, er_hbm, s_hbm, d_hbm, el_hbm, o_hbm,
              sidx, didx, ga, gb, era, erb, ala, alb, ela, elb, acc,
              gs0, gs1, ss0, ss1):
            body(G_hbm, er_hbm, s_hbm, d_hbm, el_hbm, o_hbm, None,
                 sidx, didx, (ga, gb), (era, erb), (ela, elb), (ala, alb),
                 acc, None, (gs0, gs1), (ss0, ss1))
        return [k(G, er16, src2d, dst2d, el16)]
    else:
        @deco
        def k(G_hbm, er_hbm, s_hbm, d_hbm, o_hbm, od_hbm,
              sidx, didx, ga, gb, era, erb, ala, alb, acc,
              gs0, gs1, ss0, ss1, dacc):
            body(G_hbm, er_hbm, s_hbm, d_hbm, None, o_hbm, od_hbm,
                 sidx, didx, (ga, gb), (era, erb), None, (ala, alb),
                 acc, dacc, (gs0, gs1), (ss0, ss1))
        return k(G, er16, src2d, dst2d)


def _tc_call(fn, out_shapes, *arrays):
    multi = isinstance(out_shapes, list)
    outs = out_shapes if multi else [out_shapes]

    def kern(*refs):
        ins = refs[:len(arrays)]
        os_ = refs[len(arrays):]
        res = fn(*[r[...] for r in ins])
        if not multi:
            res = (res,)
        for o, r in zip(os_, res):
            o[...] = r

    r = pl.pallas_call(
        kern,
        out_shape=[jax.ShapeDtypeStruct(s, jnp.float32) for s in outs],
    )(*arrays)
    return r if multi else r[0]


def _gcn_pre(x, norm_src, W):
    def fn(xv, nsv, Wv):
        h = (xv * nsv) @ Wv
        return jnp.concatenate([h, jnp.zeros((N_A, 12), jnp.float32)], 1)
    return _tc_call(fn, (N_A, 48), x, norm_src, W)


def _mid36(aggp, norm_dst, b, Wa, A_el, A_er):
    def fn(ap, ndv, bv, Wv, Ae, Ar):
        x2 = (ap[0] + ap[1])[:, :36] * ndv + bv
        feat = x2 @ Wv
        el = feat @ Ae
        er = feat @ Ar
        G = jnp.concatenate([feat, el, jnp.zeros((N_A, 6), jnp.float32)], 1)
        er16 = jnp.concatenate([er, jnp.zeros((N_A, 10), jnp.float32)], 1)
        return G, er16
    return _tc_call(fn, [(N_A, 48), (N_A, 16)],
                    aggp, norm_dst, b, Wa, A_el, A_er)


def _gat_post36(msgp, denp, b, R6, extra=None):
    def fn(mp, dp, bv, Rv, *rest):
        inv = 1.0 / ((dp[0] + dp[1])[:, :6] + 1e-9)
        rst = (mp[0] + mp[1])[:, :36] * (inv @ Rv) + bv
        if rest:
            rst = rst + rest[0]
        return rst
    args = (msgp, denp, b, R6) + ((extra,) if extra is not None else ())
    return _tc_call(fn, (N_A, 36), *args)


def _ma_pre(x, W, A_el, A_er):
    def fn(xv, Wv, Ae, Ar):
        feat = xv @ Wv
        el = feat @ Ae
        er = feat @ Ar
        el16 = jnp.concatenate([el, jnp.zeros((N_A, 10), jnp.float32)], 1)
        er16 = jnp.concatenate([er, jnp.zeros((N_A, 10), jnp.float32)], 1)
        return feat, el16, er16
    return _tc_call(fn, [(N_A, 192), (N_A, 16), (N_A, 16)], x, W, A_el, A_er)


def _ma_post(msgA, msgB, denp, b, R32):
    def fn(ma_, mb_, dp, bv, Rv):
        inv = 1.0 / ((dp[0] + dp[1])[:, :6] + 1e-9)
        msg = jnp.concatenate([ma_[0] + ma_[1], mb_[0] + mb_[1]], 1)
        return msg * (inv @ Rv) + bv
    return _tc_call(fn, (N_A, 192), msgA, msgB, denp, b, R32)


def _norm_stage(do0, do1, di0, di1):
    def fn(a, bb, cc, dd):
        deg_o = a + bb
        deg_i = cc + dd
        ns = lax.rsqrt(jnp.where(deg_o > 0, deg_o, 1.0))
        nd = lax.rsqrt(jnp.where(deg_i > 0, deg_i, 1.0))
        return ns[:, None], nd[:, None]
    return _tc_call(fn, [(N_A, 1), (N_A, 1)], do0, do1, di0, di1)


def _dense_stage(x, W, b):
    def fn(xv, Wv, bv):
        return xv[:N] @ Wv + bv
    return _tc_call(fn, (N, 192), x, W, b)


def _head_matrix(a):
    heads, od = a.shape
    idx = jnp.arange(heads * od)
    return jnp.zeros((heads * od, heads), jnp.float32).at[idx, idx // od].set(
        a.reshape(-1))


def kernel(features, gca1_gcn_W, gca1_gcn_b, gca1_gat_W, gca1_gat_al,
           gca1_gat_ar, gca1_gat_b, gca_gcn_W, gca_gcn_b, gca_gat_W,
           gca_gat_al, gca_gat_ar, gca_gat_b, ma_W, ma_al, ma_ar, ma_b,
           dense_W, dense_b, edge_index, num_blocks_Q, num_blocks_L):
    src = edge_index[0].astype(jnp.int32)
    dst = edge_index[1].astype(jnp.int32)
    npad = E_PAD - E
    pad_ids = (N + (jnp.arange(npad, dtype=jnp.int32) % 32)).astype(jnp.int32)
    src2d = jnp.concatenate([src, pad_ids]).reshape(NW * RW, 1, WLEN)
    dst2d = jnp.concatenate([dst, pad_ids]).reshape(NW * RW, 1, WLEN)

    do0, do1, di0, di1 = _sc_degrees(src2d, dst2d)
    norm_src, norm_dst = _norm_stage(do0, do1, di0, di1)

    x0 = jnp.pad(features, ((0, N_A - N), (0, 0)))

    A1_el = _head_matrix(gca1_gat_al)
    A1_er = _head_matrix(gca1_gat_ar)
    A_el = _head_matrix(gca_gat_al)
    A_er = _head_matrix(gca_gat_ar)
    Am_el = _head_matrix(ma_al)
    Am_er = _head_matrix(ma_ar)
    R6 = jnp.repeat(jnp.eye(6, dtype=jnp.float32), 6, axis=1)
    R32 = jnp.repeat(jnp.eye(6, dtype=jnp.float32), 32, axis=1)
    b1 = gca1_gat_b.reshape(1, 36)
    bg = gca_gat_b.reshape(1, 36)
    bm = ma_b.reshape(1, 192)

    def gca(x, Wg, bgc, Wa, Ael, Aer, ba, extra=None):
        h48 = _gcn_pre(x, norm_src, Wg)
        aggp = _sc_gcn_agg(h48, src2d, dst2d)
        G, er16 = _mid36(aggp, norm_dst, bgc.reshape(1, 36), Wa, Ael, Aer)
        msgp, denp = _sc_gat_fused(G, er16, src2d, dst2d, od=6)
        return _gat_post36(msgp, denp, ba, R6, extra)

    def gca1(x):
        return gca(x, gca1_gcn_W, gca1_gcn_b, gca1_gat_W, A1_el, A1_er, b1)

    def gcab(x, extra=None):
        return gca(x, gca_gcn_W, gca_gcn_b, gca_gat_W, A_el, A_er, bg, extra)

    def _residual_block(i, x):
        return gcab(gcab(x), extra=x)

    x = gca1(x0)
    x = jax.lax.fori_loop(0, num_blocks_Q // 2, _residual_block, x)

    feat, el16, er16 = _ma_pre(x, ma_W, Am_el, Am_er)
    msgA, denp = _sc_gat_fused(feat[:, :96], er16, src2d, dst2d,
                               el16=el16, od=32, col0=0)
    (msgB,) = _sc_gat_fused(feat[:, 96:], er16, src2d, dst2d,
                            el16=el16, od=32, col0=96, with_denom=False)
    x = _ma_post(msgA, msgB, denp, bm, R32)

    x = gca1(x)
    x = jax.lax.fori_loop(0, num_blocks_L // 2, _residual_block, x)
    return _dense_stage(x, dense_W, dense_b.reshape(1, 192))
